# Initial kernel scaffold; baseline (speedup 1.0000x reference)
import functools

import jax
import jax.numpy as jnp
from jax import lax
from jax.experimental import pallas as pl
from jax.experimental.pallas import tpu as pltpu
from jax.experimental.pallas import tpu_sc as plsc

_N = 10000
_E = 160000
_G = 64
_NB = 1000
_NC = 2
_NS = 16


def _sc_mesh():
    return plsc.VectorSubcoreMesh(
        core_axis_name="c", subcore_axis_name="s", num_cores=_NC,
        num_subcores=_NS)


_KD = 40


def _deg_body(dst_hbm, out_hbm, acc, ones_v, didx, zbuf):
    c = lax.axis_index("c")
    s = lax.axis_index("s")
    for j in range(3):
        ones_v[pl.ds(16 * j, 16)] = jnp.full((16,), 1.0, jnp.float32)
    for r in range(40):
        zbuf[pl.ds(r * 16, 16)] = jnp.zeros((16,), jnp.float32)

    @pl.when(s < 15)
    def _():
        pltpu.sync_copy(zbuf, acc.at[pl.ds(s * 640, 640)])

    @pl.when(s == 15)
    def _():
        pltpu.sync_copy(zbuf.at[pl.ds(0, 400)], acc.at[pl.ds(9600, 400)])

    plsc.subcore_barrier()

    ept = _E // (2 * _NS)
    base0 = c * (_E // 2) + s * ept

    @pl.loop(0, ept // _KD)
    def _(i):
        pltpu.sync_copy(dst_hbm.at[pl.ds(base0 + i * _KD, _KD)], didx)
        pltpu.sync_copy(ones_v.at[pl.ds(0, _KD)], acc.at[didx], add=True)

    plsc.subcore_barrier()

    @pl.when(s < 15)
    def _():
        pltpu.sync_copy(acc.at[pl.ds(s * 640, 640)],
                        out_hbm.at[pl.ds(c * _N + s * 640, 640)])

    @pl.when(s == 15)
    def _():
        pltpu.sync_copy(acc.at[pl.ds(9600, 400)],
                        out_hbm.at[pl.ds(c * _N + 9600, 400)])


def _deg(dst):
    return pl.kernel(
        _deg_body,
        out_type=jax.ShapeDtypeStruct((2 * _N,), jnp.float32),
        mesh=_sc_mesh(),
        scratch_types=[
            pltpu.VMEM_SHARED((_N,), jnp.float32),
            pltpu.VMEM((48,), jnp.float32),
            pltpu.VMEM((_KD,), jnp.int32),
            pltpu.VMEM((640,), jnp.float32),
        ],
    )(dst)


_KA = 80


def _agg_body(dh, src_hbm, dst_hbm, y_hbm, out_hbm, acc, sidx, didx, rows,
              zbuf, sem):
    c = lax.axis_index("c")
    s = lax.axis_index("s")
    coff = c * _N

    @pl.loop(0, 125)
    def _(r):
        for j in range(dh // 16):
            zbuf[r, pl.ds(j * 16, 16)] = jnp.zeros((16,), jnp.float32)

    for k in range(5):
        pltpu.sync_copy(zbuf, acc.at[pl.ds(s * 625 + k * 125, 125)])
    plsc.subcore_barrier()

    ept = _E // _NS

    @pl.loop(0, ept // _KA)
    def _(i):
        base = s * ept + i * _KA
        pltpu.sync_copy(src_hbm.at[pl.ds(base, _KA)], sidx)
        for j in range(_KA // 16):
            sidx[pl.ds(j * 16, 16)] = sidx[pl.ds(j * 16, 16)] + coff
        pltpu.async_copy(y_hbm.at[sidx], rows, sem).wait()
        pltpu.sync_copy(dst_hbm.at[pl.ds(base, _KA)], didx)
        pltpu.sync_copy(rows, acc.at[didx], add=True)

    plsc.subcore_barrier()
    for k in range(5):
        r0 = s * 625 + k * 125
        pltpu.sync_copy(acc.at[pl.ds(r0, 125)],
                        out_hbm.at[pl.ds(coff + r0, 125)])


def _agg(src, dst, y2, dh):
    return pl.kernel(
        functools.partial(_agg_body, dh),
        out_type=jax.ShapeDtypeStruct((2 * _N, dh), jnp.float32),
        mesh=_sc_mesh(),
        scratch_types=[
            pltpu.VMEM_SHARED((_N, dh), jnp.float32),
            pltpu.VMEM((_KA,), jnp.int32),
            pltpu.VMEM((_KA,), jnp.int32),
            pltpu.VMEM((_KA, dh), jnp.float32),
            pltpu.VMEM((125, dh), jnp.float32),
            pltpu.SemaphoreType.DMA,
        ],
    )(src, dst, y2)


def _tc1(degp, x, w1):
    def body(degp_ref, x_ref, w_ref, dinv_ref, y_ref):
        deg = degp_ref[0] + degp_ref[1] + 1.0
        dinv = lax.rsqrt(deg)
        xw = jnp.dot(x_ref[...], w_ref[...],
                     preferred_element_type=jnp.float32)
        y = xw * dinv
        dinv_ref[...] = dinv
        y_ref[0] = y[:, :128]
        y_ref[1] = y[:, 128:]

    return pl.pallas_call(
        body,
        grid=(_N // _NB,),
        in_specs=[
            pl.BlockSpec((2, _NB, 1), lambda i: (0, i, 0)),
            pl.BlockSpec((_NB, 512), lambda i: (i, 0)),
            pl.BlockSpec((512, 256), lambda i: (0, 0)),
        ],
        out_specs=[
            pl.BlockSpec((_NB, 1), lambda i: (i, 0)),
            pl.BlockSpec((2, _NB, 128), lambda i: (0, i, 0)),
        ],
        out_shape=[
            jax.ShapeDtypeStruct((_N, 1), jnp.float32),
            jax.ShapeDtypeStruct((2, _N, 128), jnp.float32),
        ],
    )(degp, x, w1)


def _tc_mid(agg, y, dinv, b, w, dh_in, dh_out):
    d_in = 2 * dh_in

    def body(agg_ref, y_ref, dinv_ref, b_ref, w_ref, yout_ref):
        dv = dinv_ref[...]
        bv = b_ref[...]
        h0 = jnp.maximum(dv * (agg_ref[0] + y_ref[0]) + bv[:dh_in], 0.0)
        h1 = jnp.maximum(dv * (agg_ref[1] + y_ref[1]) + bv[dh_in:], 0.0)
        h = jnp.concatenate([h0, h1], axis=1)
        xw = jnp.dot(h, w_ref[...], preferred_element_type=jnp.float32)
        yv = xw * dv
        yout_ref[0] = yv[:, :dh_out]
        yout_ref[1] = yv[:, dh_out:]

    return pl.pallas_call(
        body,
        grid=(_N // _NB,),
        in_specs=[
            pl.BlockSpec((2, _NB, dh_in), lambda i: (0, i, 0)),
            pl.BlockSpec((2, _NB, dh_in), lambda i: (0, i, 0)),
            pl.BlockSpec((_NB, 1), lambda i: (i, 0)),
            pl.BlockSpec((d_in,), lambda i: (0,)),
            pl.BlockSpec((d_in, 2 * dh_out), lambda i: (0, 0)),
        ],
        out_specs=pl.BlockSpec((2, _NB, dh_out), lambda i: (0, i, 0)),
        out_shape=jax.ShapeDtypeStruct((2, _N, dh_out), jnp.float32),
    )(agg, y, dinv, b, w)


def _tc5(agg, y, dinv, b4, batch2, lw1, lb1, lw2, lb2):
    nblk = _N // _NB

    def body(agg_ref, y_ref, dinv_ref, b_ref, batch_ref, lw1_ref, lb1_ref,
             lw2_ref, lb2_ref, out_ref, sums, cnt):
        i = pl.program_id(0)
        dv = dinv_ref[...]
        bv = b_ref[...]
        h0 = dv * (agg_ref[0] + y_ref[0]) + bv[:16]
        h1 = dv * (agg_ref[1] + y_ref[1]) + bv[16:]
        h = jnp.concatenate([h0, h1], axis=1)
        gi = lax.broadcasted_iota(jnp.int32, (_NB, _G), 1)
        oh = (batch_ref[...] == gi).astype(jnp.float32)
        ps = lax.dot_general(oh, h, (((0,), (0,)), ((), ())),
                             preferred_element_type=jnp.float32)
        pc = lax.dot_general(oh, jnp.ones((_NB, 1), jnp.float32),
                             (((0,), (0,)), ((), ())),
                             preferred_element_type=jnp.float32)

        @pl.when(i == 0)
        def _():
            sums[...] = ps
            cnt[...] = pc
            out_ref[...] = jnp.zeros_like(out_ref)

        @pl.when(i > 0)
        def _():
            sums[...] = sums[...] + ps
            cnt[...] = cnt[...] + pc

        @pl.when(i == nblk - 1)
        def _():
            pooled = sums[...] / jnp.maximum(cnt[...], 1.0)
            z = jnp.maximum(
                jnp.dot(pooled, lw1_ref[...],
                        preferred_element_type=jnp.float32) + lb1_ref[...],
                0.0)
            out_ref[...] = jnp.dot(
                z, lw2_ref[...], preferred_element_type=jnp.float32) \
                + lb2_ref[...]

    return pl.pallas_call(
        body,
        grid=(nblk,),
        in_specs=[
            pl.BlockSpec((2, _NB, 16), lambda i: (0, i, 0)),
            pl.BlockSpec((2, _NB, 16), lambda i: (0, i, 0)),
            pl.BlockSpec((_NB, 1), lambda i: (i, 0)),
            pl.BlockSpec((32,), lambda i: (0,)),
            pl.BlockSpec((_NB, 1), lambda i: (i, 0)),
            pl.BlockSpec((32, 16), lambda i: (0, 0)),
            pl.BlockSpec((16,), lambda i: (0,)),
            pl.BlockSpec((16, 2), lambda i: (0, 0)),
            pl.BlockSpec((2,), lambda i: (0,)),
        ],
        out_specs=pl.BlockSpec((_G, 2), lambda i: (0, 0)),
        out_shape=jax.ShapeDtypeStruct((_G, 2), jnp.float32),
        scratch_shapes=[
            pltpu.VMEM((_G, 32), jnp.float32),
            pltpu.VMEM((_G, 1), jnp.float32),
        ],
    )(agg, y, dinv, b4, batch2, lw1, lb1, lw2, lb2)


def kernel(x, edge_index, batch, w1, b1, w2, b2, w3, b3, w4, b4, lw1, lb1,
           lw2, lb2):
    src = edge_index[0]
    dst = edge_index[1]
    degp = _deg(dst).reshape(2, _N, 1)
    dinv, y1 = _tc1(degp, x, w1)
    agg1 = _agg(src, dst, y1.reshape(2 * _N, 128), 128).reshape(2, _N, 128)
    y2 = _tc_mid(agg1, y1, dinv, b1, w2, 128, 64)
    agg2 = _agg(src, dst, y2.reshape(2 * _N, 64), 64).reshape(2, _N, 64)
    y3 = _tc_mid(agg2, y2, dinv, b2, w3, 64, 32)
    agg3 = _agg(src, dst, y3.reshape(2 * _N, 32), 32).reshape(2, _N, 32)
    y4 = _tc_mid(agg3, y3, dinv, b3, w4, 32, 16)
    agg4 = _agg(src, dst, y4.reshape(2 * _N, 16), 16).reshape(2, _N, 16)
    return _tc5(agg4, y4, dinv, b4, batch[:, None], lw1, lb1, lw2, lb2)


# speedup vs baseline: 6.3044x; 6.3044x over previous
import functools

import jax
import jax.numpy as jnp
from jax import lax
from jax.experimental import pallas as pl
from jax.experimental.pallas import tpu as pltpu
from jax.experimental.pallas import tpu_sc as plsc

_N = 10000
_E = 160000
_G = 64
_NB = 1000
_NC = 2
_NS = 16


def _sc_mesh():
    return plsc.VectorSubcoreMesh(
        core_axis_name="c", subcore_axis_name="s", num_cores=_NC,
        num_subcores=_NS)


_KD = 40


def _deg_body(dst_hbm, out_hbm, acc, ones_v, didx, zbuf):
    c = lax.axis_index("c")
    s = lax.axis_index("s")
    for j in range(3):
        ones_v[pl.ds(16 * j, 16)] = jnp.full((16,), 1.0, jnp.float32)
    for r in range(40):
        zbuf[pl.ds(r * 16, 16)] = jnp.zeros((16,), jnp.float32)

    @pl.when(s < 15)
    def _():
        pltpu.sync_copy(zbuf, acc.at[pl.ds(s * 640, 640)])

    @pl.when(s == 15)
    def _():
        pltpu.sync_copy(zbuf.at[pl.ds(0, 400)], acc.at[pl.ds(9600, 400)])

    plsc.subcore_barrier()

    ept = _E // (2 * _NS)
    base0 = c * (_E // 2) + s * ept

    @pl.loop(0, ept // _KD)
    def _(i):
        pltpu.sync_copy(dst_hbm.at[pl.ds(base0 + i * _KD, _KD)], didx)
        pltpu.sync_copy(ones_v.at[pl.ds(0, _KD)], acc.at[didx], add=True)

    plsc.subcore_barrier()

    @pl.when(s < 15)
    def _():
        pltpu.sync_copy(acc.at[pl.ds(s * 640, 640)], zbuf)
        pltpu.sync_copy(zbuf, out_hbm.at[pl.ds(c * _N + s * 640, 640)])

    @pl.when(s == 15)
    def _():
        pltpu.sync_copy(acc.at[pl.ds(9600, 400)], zbuf.at[pl.ds(0, 400)])
        pltpu.sync_copy(zbuf.at[pl.ds(0, 400)],
                        out_hbm.at[pl.ds(c * _N + 9600, 400)])


def _deg(dst):
    return pl.kernel(
        _deg_body,
        out_type=jax.ShapeDtypeStruct((2 * _N,), jnp.float32),
        mesh=_sc_mesh(),
        scratch_types=[
            pltpu.VMEM_SHARED((_N,), jnp.float32),
            pltpu.VMEM((48,), jnp.float32),
            pltpu.VMEM((_KD,), jnp.int32),
            pltpu.VMEM((640,), jnp.float32),
        ],
    )(dst)


_KA = 80


def _agg_body(dh, src_hbm, dst_hbm, y_hbm, out_hbm, acc, sidx, didx, rows,
              zbuf, sem):
    c = lax.axis_index("c")
    s = lax.axis_index("s")
    coff = c * _N

    @pl.loop(0, 80)
    def _(r):
        for j in range(dh // 16):
            zbuf[r, pl.ds(j * 16, 16)] = jnp.zeros((16,), jnp.float32)

    @pl.when(s < 15)
    def _():
        for k in range(8):
            pltpu.sync_copy(zbuf, acc.at[pl.ds(s * 640 + k * 80, 80)])

    @pl.when(s == 15)
    def _():
        for k in range(5):
            pltpu.sync_copy(zbuf, acc.at[pl.ds(9600 + k * 80, 80)])

    plsc.subcore_barrier()

    ept = _E // _NS

    @pl.loop(0, ept // _KA)
    def _(i):
        base = s * ept + i * _KA
        pltpu.sync_copy(src_hbm.at[pl.ds(base, _KA)], sidx)
        for j in range(_KA // 16):
            sidx[pl.ds(j * 16, 16)] = sidx[pl.ds(j * 16, 16)] + coff
        pltpu.async_copy(y_hbm.at[sidx], rows, sem).wait()
        pltpu.sync_copy(dst_hbm.at[pl.ds(base, _KA)], didx)
        pltpu.sync_copy(rows, acc.at[didx], add=True)

    plsc.subcore_barrier()

    @pl.when(s < 15)
    def _():
        for k in range(8):
            r0 = s * 640 + k * 80
            pltpu.sync_copy(acc.at[pl.ds(r0, 80)], zbuf)
            pltpu.sync_copy(zbuf, out_hbm.at[pl.ds(coff + r0, 80)])

    @pl.when(s == 15)
    def _():
        for k in range(5):
            r0 = 9600 + k * 80
            pltpu.sync_copy(acc.at[pl.ds(r0, 80)], zbuf)
            pltpu.sync_copy(zbuf, out_hbm.at[pl.ds(coff + r0, 80)])


def _agg(src, dst, y2, dh):
    return pl.kernel(
        functools.partial(_agg_body, dh),
        out_type=jax.ShapeDtypeStruct((2 * _N, dh), jnp.float32),
        mesh=_sc_mesh(),
        scratch_types=[
            pltpu.VMEM_SHARED((_N, dh), jnp.float32),
            pltpu.VMEM((_KA,), jnp.int32),
            pltpu.VMEM((_KA,), jnp.int32),
            pltpu.VMEM((_KA, dh), jnp.float32),
            pltpu.VMEM((80, dh), jnp.float32),
            pltpu.SemaphoreType.DMA,
        ],
        compiler_params=pltpu.CompilerParams(use_tc_tiling_on_sc=False),
    )(src, dst, y2)


def _tc1(degp, x, w1):
    def body(degp_ref, x_ref, w_ref, dinv_ref, y_ref):
        deg = degp_ref[0] + degp_ref[1] + 1.0
        dinv = lax.rsqrt(deg)
        xw = jnp.dot(x_ref[...], w_ref[...],
                     preferred_element_type=jnp.float32)
        y = xw * dinv
        dinv_ref[...] = dinv
        y_ref[0] = y[:, :128]
        y_ref[1] = y[:, 128:]

    return pl.pallas_call(
        body,
        grid=(_N // _NB,),
        in_specs=[
            pl.BlockSpec((2, _NB, 1), lambda i: (0, i, 0)),
            pl.BlockSpec((_NB, 512), lambda i: (i, 0)),
            pl.BlockSpec((512, 256), lambda i: (0, 0)),
        ],
        out_specs=[
            pl.BlockSpec((_NB, 1), lambda i: (i, 0)),
            pl.BlockSpec((2, _NB, 128), lambda i: (0, i, 0)),
        ],
        out_shape=[
            jax.ShapeDtypeStruct((_N, 1), jnp.float32),
            jax.ShapeDtypeStruct((2, _N, 128), jnp.float32),
        ],
    )(degp, x, w1)


def _tc_mid(agg, y, dinv, b, w, dh_in, dh_out):
    d_in = 2 * dh_in

    def body(agg_ref, y_ref, dinv_ref, b_ref, w_ref, yout_ref):
        dv = dinv_ref[...]
        bv = b_ref[...]
        h0 = jnp.maximum(dv * (agg_ref[0] + y_ref[0]) + bv[:dh_in], 0.0)
        h1 = jnp.maximum(dv * (agg_ref[1] + y_ref[1]) + bv[dh_in:], 0.0)
        h = jnp.concatenate([h0, h1], axis=1)
        xw = jnp.dot(h, w_ref[...], preferred_element_type=jnp.float32)
        yv = xw * dv
        yout_ref[0] = yv[:, :dh_out]
        yout_ref[1] = yv[:, dh_out:]

    return pl.pallas_call(
        body,
        grid=(_N // _NB,),
        in_specs=[
            pl.BlockSpec((2, _NB, dh_in), lambda i: (0, i, 0)),
            pl.BlockSpec((2, _NB, dh_in), lambda i: (0, i, 0)),
            pl.BlockSpec((_NB, 1), lambda i: (i, 0)),
            pl.BlockSpec((d_in,), lambda i: (0,)),
            pl.BlockSpec((d_in, 2 * dh_out), lambda i: (0, 0)),
        ],
        out_specs=pl.BlockSpec((2, _NB, dh_out), lambda i: (0, i, 0)),
        out_shape=jax.ShapeDtypeStruct((2, _N, dh_out), jnp.float32),
    )(agg, y, dinv, b, w)


def _tc5(agg, y, dinv, b4, batch2, lw1, lb1, lw2, lb2):
    nblk = _N // _NB

    def body(agg_ref, y_ref, dinv_ref, b_ref, batch_ref, lw1_ref, lb1_ref,
             lw2_ref, lb2_ref, out_ref, sums, cnt):
        i = pl.program_id(0)
        dv = dinv_ref[...]
        bv = b_ref[...]
        h0 = dv * (agg_ref[0] + y_ref[0]) + bv[:16]
        h1 = dv * (agg_ref[1] + y_ref[1]) + bv[16:]
        h = jnp.concatenate([h0, h1], axis=1)
        gi = lax.broadcasted_iota(jnp.int32, (_NB, _G), 1)
        oh = (batch_ref[...] == gi).astype(jnp.float32)
        ps = lax.dot_general(oh, h, (((0,), (0,)), ((), ())),
                             preferred_element_type=jnp.float32)
        pc = lax.dot_general(oh, jnp.ones((_NB, 1), jnp.float32),
                             (((0,), (0,)), ((), ())),
                             preferred_element_type=jnp.float32)

        @pl.when(i == 0)
        def _():
            sums[...] = ps
            cnt[...] = pc
            out_ref[...] = jnp.zeros_like(out_ref)

        @pl.when(i > 0)
        def _():
            sums[...] = sums[...] + ps
            cnt[...] = cnt[...] + pc

        @pl.when(i == nblk - 1)
        def _():
            pooled = sums[...] / jnp.maximum(cnt[...], 1.0)
            z = jnp.maximum(
                jnp.dot(pooled, lw1_ref[...],
                        preferred_element_type=jnp.float32) + lb1_ref[...],
                0.0)
            out_ref[...] = jnp.dot(
                z, lw2_ref[...], preferred_element_type=jnp.float32) \
                + lb2_ref[...]

    return pl.pallas_call(
        body,
        grid=(nblk,),
        in_specs=[
            pl.BlockSpec((2, _NB, 16), lambda i: (0, i, 0)),
            pl.BlockSpec((2, _NB, 16), lambda i: (0, i, 0)),
            pl.BlockSpec((_NB, 1), lambda i: (i, 0)),
            pl.BlockSpec((32,), lambda i: (0,)),
            pl.BlockSpec((_NB, 1), lambda i: (i, 0)),
            pl.BlockSpec((32, 16), lambda i: (0, 0)),
            pl.BlockSpec((16,), lambda i: (0,)),
            pl.BlockSpec((16, 2), lambda i: (0, 0)),
            pl.BlockSpec((2,), lambda i: (0,)),
        ],
        out_specs=pl.BlockSpec((_G, 2), lambda i: (0, 0)),
        out_shape=jax.ShapeDtypeStruct((_G, 2), jnp.float32),
        scratch_shapes=[
            pltpu.VMEM((_G, 32), jnp.float32),
            pltpu.VMEM((_G, 1), jnp.float32),
        ],
    )(agg, y, dinv, b4, batch2, lw1, lb1, lw2, lb2)


def kernel(x, edge_index, batch, w1, b1, w2, b2, w3, b3, w4, b4, lw1, lb1,
           lw2, lb2):
    src = edge_index[0]
    dst = edge_index[1]
    degp = _deg(dst).reshape(2, _N, 1)
    dinv, y1 = _tc1(degp, x, w1)
    agg1 = _agg(src, dst, y1.reshape(2 * _N, 128), 128).reshape(2, _N, 128)
    y2 = _tc_mid(agg1, y1, dinv, b1, w2, 128, 64)
    agg2 = _agg(src, dst, y2.reshape(2 * _N, 64), 64).reshape(2, _N, 64)
    y3 = _tc_mid(agg2, y2, dinv, b2, w3, 64, 32)
    agg3 = _agg(src, dst, y3.reshape(2 * _N, 32), 32).reshape(2, _N, 32)
    y4 = _tc_mid(agg3, y3, dinv, b3, w4, 32, 16)
    agg4 = _agg(src, dst, y4.reshape(2 * _N, 16), 16).reshape(2, _N, 16)
    return _tc5(agg4, y4, dinv, b4, batch[:, None], lw1, lb1, lw2, lb2)


# speedup vs baseline: 12.3578x; 1.9602x over previous
import functools

import jax
import jax.numpy as jnp
from jax import lax
from jax.experimental import pallas as pl
from jax.experimental.pallas import tpu as pltpu
from jax.experimental.pallas import tpu_sc as plsc

_N = 10000
_E = 160000
_G = 64
_NB = 1000
_NC = 2
_NS = 16


def _sc_mesh():
    return plsc.VectorSubcoreMesh(
        core_axis_name="c", subcore_axis_name="s", num_cores=_NC,
        num_subcores=_NS)


_KD = 40


def _deg_body(dst_hbm, out_hbm, acc, ones_v, didx, zbuf):
    c = lax.axis_index("c")
    s = lax.axis_index("s")
    for j in range(3):
        ones_v[pl.ds(16 * j, 16)] = jnp.full((16,), 1.0, jnp.float32)
    for r in range(40):
        zbuf[pl.ds(r * 16, 16)] = jnp.zeros((16,), jnp.float32)

    @pl.when(s < 15)
    def _():
        pltpu.sync_copy(zbuf, acc.at[pl.ds(s * 640, 640)])

    @pl.when(s == 15)
    def _():
        pltpu.sync_copy(zbuf.at[pl.ds(0, 400)], acc.at[pl.ds(9600, 400)])

    plsc.subcore_barrier()

    ept = _E // (2 * _NS)
    base0 = c * (_E // 2) + s * ept

    @pl.loop(0, ept // _KD)
    def _(i):
        pltpu.sync_copy(dst_hbm.at[pl.ds(base0 + i * _KD, _KD)], didx)
        pltpu.sync_copy(ones_v.at[pl.ds(0, _KD)], acc.at[didx], add=True)

    plsc.subcore_barrier()

    @pl.when(s < 15)
    def _():
        pltpu.sync_copy(acc.at[pl.ds(s * 640, 640)], zbuf)
        pltpu.sync_copy(zbuf, out_hbm.at[pl.ds(c * _N + s * 640, 640)])

    @pl.when(s == 15)
    def _():
        pltpu.sync_copy(acc.at[pl.ds(9600, 400)], zbuf.at[pl.ds(0, 400)])
        pltpu.sync_copy(zbuf.at[pl.ds(0, 400)],
                        out_hbm.at[pl.ds(c * _N + 9600, 400)])


def _deg(dst):
    return pl.kernel(
        _deg_body,
        out_type=jax.ShapeDtypeStruct((2 * _N,), jnp.float32),
        mesh=_sc_mesh(),
        scratch_types=[
            pltpu.VMEM_SHARED((_N,), jnp.float32),
            pltpu.VMEM((48,), jnp.float32),
            pltpu.VMEM((_KD,), jnp.int32),
            pltpu.VMEM((640,), jnp.float32),
        ],
    )(dst)


_KA = 125
_NCH = 80


_GC = 8
_NG = _NCH // _GC


def _agg_body(dh, src4_hbm, dst3_hbm, y_hbm, out_hbm, acc, gsi, gdi,
              b0, b1, sg, ss0, ss1):
    c = lax.axis_index("c")
    s = lax.axis_index("s")
    coff = c * _N

    bufs = (b0, b1)
    sss = (ss0, ss1)

    pltpu.sync_copy(src4_hbm.at[c, s].at[pl.ds(0, _GC)], gsi.at[0])
    pltpu.sync_copy(dst3_hbm.at[s].at[pl.ds(0, _GC)], gdi.at[0])

    @pl.loop(0, 80)
    def _(r):
        for j in range(dh // 16):
            b1[r, pl.ds(j * 16, 16)] = jnp.zeros((16,), jnp.float32)

    pltpu.async_copy(y_hbm.at[gsi.at[0, 0]], b0, sg)

    zsrc = b1.at[pl.ds(0, 80)]

    @pl.when(s < 15)
    def _():
        for k in range(8):
            pltpu.sync_copy(zsrc, acc.at[pl.ds(s * 640 + k * 80, 80)])

    @pl.when(s == 15)
    def _():
        for k in range(5):
            pltpu.sync_copy(zsrc, acc.at[pl.ds(9600 + k * 80, 80)])

    plsc.subcore_barrier()

    def gwait(qq, k, buf):
        pltpu.make_async_copy(y_hbm.at[gsi.at[qq, k]], buf, sg).wait()

    def swait(qq, k, buf, sem):
        pltpu.make_async_copy(buf, acc.at[gdi.at[qq, k]], sem).wait()

    @pl.loop(0, _NG // 2)
    def _(gp):
        for qq in range(2):
            g = 2 * gp + qq
            for k in range(_GC):
                x = bufs[k % 2]
                o = bufs[1 - k % 2]
                gwait(qq, k, x)
                pltpu.async_copy(x, acc.at[gdi.at[qq, k]], sss[k % 2],
                                 add=True)
                def _wprev():
                    swait(qq, k, o, sss[1 - k % 2])

                if k == 0:
                    pl.when(g > 0)(_wprev)
                else:
                    _wprev()

                if k == 1:
                    def _pref():
                        pltpu.sync_copy(
                            src4_hbm.at[c, s].at[pl.ds((g + 1) * _GC, _GC)],
                            gsi.at[1 - qq])
                        pltpu.sync_copy(
                            dst3_hbm.at[s].at[pl.ds((g + 1) * _GC, _GC)],
                            gdi.at[1 - qq])

                    pl.when(g < _NG - 1)(_pref)

                if k < _GC - 1:
                    pltpu.async_copy(y_hbm.at[gsi.at[qq, k + 1]], o, sg)
                else:
                    def _gnext():
                        pltpu.async_copy(y_hbm.at[gsi.at[1 - qq, 0]], o, sg)

                    pl.when(g < _NG - 1)(_gnext)

    swait(1, _GC - 1, b1, ss1)

    plsc.subcore_barrier()

    stg = b0.at[pl.ds(0, 80)]

    @pl.when(s < 15)
    def _():
        for k in range(8):
            r0 = s * 640 + k * 80
            pltpu.sync_copy(acc.at[pl.ds(r0, 80)], stg)
            pltpu.sync_copy(stg, out_hbm.at[pl.ds(coff + r0, 80)])

    @pl.when(s == 15)
    def _():
        for k in range(5):
            r0 = 9600 + k * 80
            pltpu.sync_copy(acc.at[pl.ds(r0, 80)], stg)
            pltpu.sync_copy(stg, out_hbm.at[pl.ds(coff + r0, 80)])


def _agg(src4, dst3, y2, dh):
    return pl.kernel(
        functools.partial(_agg_body, dh),
        out_type=jax.ShapeDtypeStruct((2 * _N, dh), jnp.float32),
        mesh=_sc_mesh(),
        scratch_types=[
            pltpu.VMEM_SHARED((_N, dh), jnp.float32),
            pltpu.VMEM((2, _GC, _KA), jnp.int32),
            pltpu.VMEM((2, _GC, _KA), jnp.int32),
            pltpu.VMEM((_KA, dh), jnp.float32),
            pltpu.VMEM((_KA, dh), jnp.float32),
            pltpu.SemaphoreType.DMA,
            pltpu.SemaphoreType.DMA,
            pltpu.SemaphoreType.DMA,
        ],
        compiler_params=pltpu.CompilerParams(use_tc_tiling_on_sc=False),
    )(src4, dst3, y2)


def _tc1(degp, x, w1):
    def body(degp_ref, x_ref, w_ref, dinv_ref, y_ref):
        deg = degp_ref[0] + degp_ref[1] + 1.0
        dinv = lax.rsqrt(deg)
        xw = jnp.dot(x_ref[...], w_ref[...],
                     preferred_element_type=jnp.float32)
        y = xw * dinv
        dinv_ref[...] = dinv
        y_ref[0] = y[:, :128]
        y_ref[1] = y[:, 128:]

    return pl.pallas_call(
        body,
        grid=(_N // _NB,),
        in_specs=[
            pl.BlockSpec((2, _NB, 1), lambda i: (0, i, 0)),
            pl.BlockSpec((_NB, 512), lambda i: (i, 0)),
            pl.BlockSpec((512, 256), lambda i: (0, 0)),
        ],
        out_specs=[
            pl.BlockSpec((_NB, 1), lambda i: (i, 0)),
            pl.BlockSpec((2, _NB, 128), lambda i: (0, i, 0)),
        ],
        out_shape=[
            jax.ShapeDtypeStruct((_N, 1), jnp.float32),
            jax.ShapeDtypeStruct((2, _N, 128), jnp.float32),
        ],
    )(degp, x, w1)


def _tc_mid(agg, y, dinv, b, w, dh_in, dh_out):
    d_in = 2 * dh_in

    def body(agg_ref, y_ref, dinv_ref, b_ref, w_ref, yout_ref):
        dv = dinv_ref[...]
        bv = b_ref[...]
        h0 = jnp.maximum(dv * (agg_ref[0] + y_ref[0]) + bv[:dh_in], 0.0)
        h1 = jnp.maximum(dv * (agg_ref[1] + y_ref[1]) + bv[dh_in:], 0.0)
        h = jnp.concatenate([h0, h1], axis=1)
        xw = jnp.dot(h, w_ref[...], preferred_element_type=jnp.float32)
        yv = xw * dv
        yout_ref[0] = yv[:, :dh_out]
        yout_ref[1] = yv[:, dh_out:]

    return pl.pallas_call(
        body,
        grid=(_N // _NB,),
        in_specs=[
            pl.BlockSpec((2, _NB, dh_in), lambda i: (0, i, 0)),
            pl.BlockSpec((2, _NB, dh_in), lambda i: (0, i, 0)),
            pl.BlockSpec((_NB, 1), lambda i: (i, 0)),
            pl.BlockSpec((d_in,), lambda i: (0,)),
            pl.BlockSpec((d_in, 2 * dh_out), lambda i: (0, 0)),
        ],
        out_specs=pl.BlockSpec((2, _NB, dh_out), lambda i: (0, i, 0)),
        out_shape=jax.ShapeDtypeStruct((2, _N, dh_out), jnp.float32),
    )(agg, y, dinv, b, w)


def _tc5(agg, y, dinv, b4, batch2, lw1, lb1, lw2, lb2):
    nblk = _N // _NB

    def body(agg_ref, y_ref, dinv_ref, b_ref, batch_ref, lw1_ref, lb1_ref,
             lw2_ref, lb2_ref, out_ref, sums, cnt):
        i = pl.program_id(0)
        dv = dinv_ref[...]
        bv = b_ref[...]
        h0 = dv * (agg_ref[0] + y_ref[0]) + bv[:16]
        h1 = dv * (agg_ref[1] + y_ref[1]) + bv[16:]
        h = jnp.concatenate([h0, h1], axis=1)
        gi = lax.broadcasted_iota(jnp.int32, (_NB, _G), 1)
        oh = (batch_ref[...] == gi).astype(jnp.float32)
        ps = lax.dot_general(oh, h, (((0,), (0,)), ((), ())),
                             preferred_element_type=jnp.float32)
        pc = lax.dot_general(oh, jnp.ones((_NB, 1), jnp.float32),
                             (((0,), (0,)), ((), ())),
                             preferred_element_type=jnp.float32)

        @pl.when(i == 0)
        def _():
            sums[...] = ps
            cnt[...] = pc
            out_ref[...] = jnp.zeros_like(out_ref)

        @pl.when(i > 0)
        def _():
            sums[...] = sums[...] + ps
            cnt[...] = cnt[...] + pc

        @pl.when(i == nblk - 1)
        def _():
            pooled = sums[...] / jnp.maximum(cnt[...], 1.0)
            z = jnp.maximum(
                jnp.dot(pooled, lw1_ref[...],
                        preferred_element_type=jnp.float32) + lb1_ref[...],
                0.0)
            out_ref[...] = jnp.dot(
                z, lw2_ref[...], preferred_element_type=jnp.float32) \
                + lb2_ref[...]

    return pl.pallas_call(
        body,
        grid=(nblk,),
        in_specs=[
            pl.BlockSpec((2, _NB, 16), lambda i: (0, i, 0)),
            pl.BlockSpec((2, _NB, 16), lambda i: (0, i, 0)),
            pl.BlockSpec((_NB, 1), lambda i: (i, 0)),
            pl.BlockSpec((32,), lambda i: (0,)),
            pl.BlockSpec((_NB, 1), lambda i: (i, 0)),
            pl.BlockSpec((32, 16), lambda i: (0, 0)),
            pl.BlockSpec((16,), lambda i: (0,)),
            pl.BlockSpec((16, 2), lambda i: (0, 0)),
            pl.BlockSpec((2,), lambda i: (0,)),
        ],
        out_specs=pl.BlockSpec((_G, 2), lambda i: (0, 0)),
        out_shape=jax.ShapeDtypeStruct((_G, 2), jnp.float32),
        scratch_shapes=[
            pltpu.VMEM((_G, 32), jnp.float32),
            pltpu.VMEM((_G, 1), jnp.float32),
        ],
    )(agg, y, dinv, b4, batch2, lw1, lb1, lw2, lb2)


def kernel(x, edge_index, batch, w1, b1, w2, b2, w3, b3, w4, b4, lw1, lb1,
           lw2, lb2):
    src = edge_index[0]
    dst = edge_index[1]
    src4 = jnp.stack([src, src + _N]).reshape(2, _NS, _NCH, _KA)
    dst3 = dst.reshape(_NS, _NCH, _KA)
    degp = _deg(dst).reshape(2, _N, 1)
    dinv, y1 = _tc1(degp, x, w1)
    agg1 = _agg(src4, dst3, y1.reshape(2 * _N, 128), 128).reshape(2, _N, 128)
    y2 = _tc_mid(agg1, y1, dinv, b1, w2, 128, 64)
    agg2 = _agg(src4, dst3, y2.reshape(2 * _N, 64), 64).reshape(2, _N, 64)
    y3 = _tc_mid(agg2, y2, dinv, b2, w3, 64, 32)
    agg3 = _agg(src4, dst3, y3.reshape(2 * _N, 32), 32).reshape(2, _N, 32)
    y4 = _tc_mid(agg3, y3, dinv, b3, w4, 32, 16)
    agg4 = _agg(src4, dst3, y4.reshape(2 * _N, 16), 16).reshape(2, _N, 16)
    return _tc5(agg4, y4, dinv, b4, batch[:, None], lw1, lb1, lw2, lb2)


# speedup vs baseline: 14.2533x; 1.1534x over previous
import functools

import jax
import jax.numpy as jnp
from jax import lax
from jax.experimental import pallas as pl
from jax.experimental.pallas import tpu as pltpu
from jax.experimental.pallas import tpu_sc as plsc

_N = 10000
_E = 160000
_G = 64
_NB = 1000
_NC = 2
_NS = 16


def _sc_mesh():
    return plsc.VectorSubcoreMesh(
        core_axis_name="c", subcore_axis_name="s", num_cores=_NC,
        num_subcores=_NS)


_KD = 40


def _deg_body(dst_hbm, out_hbm, acc, ones_v, didx, zbuf):
    c = lax.axis_index("c")
    s = lax.axis_index("s")
    for j in range(3):
        ones_v[pl.ds(16 * j, 16)] = jnp.full((16,), 1.0, jnp.float32)
    for r in range(40):
        zbuf[pl.ds(r * 16, 16)] = jnp.zeros((16,), jnp.float32)

    @pl.when(s < 15)
    def _():
        pltpu.sync_copy(zbuf, acc.at[pl.ds(s * 640, 640)])

    @pl.when(s == 15)
    def _():
        pltpu.sync_copy(zbuf.at[pl.ds(0, 400)], acc.at[pl.ds(9600, 400)])

    plsc.subcore_barrier()

    ept = _E // (2 * _NS)
    base0 = c * (_E // 2) + s * ept

    @pl.loop(0, ept // _KD)
    def _(i):
        pltpu.sync_copy(dst_hbm.at[pl.ds(base0 + i * _KD, _KD)], didx)
        pltpu.sync_copy(ones_v.at[pl.ds(0, _KD)], acc.at[didx], add=True)

    plsc.subcore_barrier()

    @pl.when(s < 15)
    def _():
        pltpu.sync_copy(acc.at[pl.ds(s * 640, 640)], zbuf)
        pltpu.sync_copy(zbuf, out_hbm.at[pl.ds(c * _N + s * 640, 640)])

    @pl.when(s == 15)
    def _():
        pltpu.sync_copy(acc.at[pl.ds(9600, 400)], zbuf.at[pl.ds(0, 400)])
        pltpu.sync_copy(zbuf.at[pl.ds(0, 400)],
                        out_hbm.at[pl.ds(c * _N + 9600, 400)])


def _deg(dst):
    return pl.kernel(
        _deg_body,
        out_type=jax.ShapeDtypeStruct((2 * _N,), jnp.float32),
        mesh=_sc_mesh(),
        scratch_types=[
            pltpu.VMEM_SHARED((_N,), jnp.float32),
            pltpu.VMEM((48,), jnp.float32),
            pltpu.VMEM((_KD,), jnp.int32),
            pltpu.VMEM((640,), jnp.float32),
        ],
    )(dst)


_KA = 125
_NCH = 80


_GC = 8
_NG = _NCH // _GC


def _agg_body(dh, src4_hbm, dst3_hbm, y_hbm, out_hbm, acc, gsi, gdi,
              b0, b1, sg, ss0, ss1):
    c = lax.axis_index("c")
    s = lax.axis_index("s")
    coff = c * _N

    bufs = (b0, b1)
    sss = (ss0, ss1)

    pltpu.sync_copy(src4_hbm.at[c, s].at[pl.ds(0, _GC)], gsi.at[0])
    pltpu.sync_copy(dst3_hbm.at[s].at[pl.ds(0, _GC)], gdi.at[0])

    @pl.loop(0, 80)
    def _(r):
        for j in range(dh // 16):
            b1[r, pl.ds(j * 16, 16)] = jnp.zeros((16,), jnp.float32)

    pltpu.async_copy(y_hbm.at[gsi.at[0, 0]], b0, sg)

    zsrc = b1.at[pl.ds(0, 80)]

    @pl.when(s < 15)
    def _():
        for k in range(8):
            pltpu.sync_copy(zsrc, acc.at[pl.ds(s * 640 + k * 80, 80)])

    @pl.when(s == 15)
    def _():
        for k in range(5):
            pltpu.sync_copy(zsrc, acc.at[pl.ds(9600 + k * 80, 80)])

    plsc.subcore_barrier()

    def gwait(qq, k, buf):
        pltpu.make_async_copy(y_hbm.at[gsi.at[qq, k]], buf, sg).wait()

    def swait(qq, k, buf, sem):
        pltpu.make_async_copy(buf, acc.at[gdi.at[qq, k]], sem).wait()

    @pl.loop(0, _NG // 2)
    def _(gp):
        for qq in range(2):
            g = 2 * gp + qq
            for k in range(_GC):
                x = bufs[k % 2]
                o = bufs[1 - k % 2]
                gwait(qq, k, x)
                pltpu.async_copy(x, acc.at[gdi.at[qq, k]], sss[k % 2],
                                 add=True)
                def _wprev():
                    swait(qq, k, o, sss[1 - k % 2])

                if k == 0:
                    pl.when(g > 0)(_wprev)
                else:
                    _wprev()

                if k == 1:
                    def _pref():
                        pltpu.sync_copy(
                            src4_hbm.at[c, s].at[pl.ds((g + 1) * _GC, _GC)],
                            gsi.at[1 - qq])
                        pltpu.sync_copy(
                            dst3_hbm.at[s].at[pl.ds((g + 1) * _GC, _GC)],
                            gdi.at[1 - qq])

                    pl.when(g < _NG - 1)(_pref)

                if k < _GC - 1:
                    pltpu.async_copy(y_hbm.at[gsi.at[qq, k + 1]], o, sg)
                else:
                    def _gnext():
                        pltpu.async_copy(y_hbm.at[gsi.at[1 - qq, 0]], o, sg)

                    pl.when(g < _NG - 1)(_gnext)

    swait(1, _GC - 1, b1, ss1)

    plsc.subcore_barrier()

    stg = b0.at[pl.ds(0, 80)]

    @pl.when(s < 15)
    def _():
        for k in range(8):
            r0 = s * 640 + k * 80
            pltpu.sync_copy(acc.at[pl.ds(r0, 80)], stg)
            pltpu.sync_copy(stg, out_hbm.at[pl.ds(coff + r0, 80)])

    @pl.when(s == 15)
    def _():
        for k in range(5):
            r0 = 9600 + k * 80
            pltpu.sync_copy(acc.at[pl.ds(r0, 80)], stg)
            pltpu.sync_copy(stg, out_hbm.at[pl.ds(coff + r0, 80)])


def _agg_body4(dh, src4_hbm, dst3_hbm, y_hbm, out_hbm, acc, gsi, gdi,
               b0, b1, b2, b3, sg0, sg1, sg2, sg3, ss0, ss1, ss2, ss3):
    c = lax.axis_index("c")
    s = lax.axis_index("s")
    coff = c * _N

    bufs = (b0, b1, b2, b3)
    sgs = (sg0, sg1, sg2, sg3)
    sss = (ss0, ss1, ss2, ss3)

    pltpu.sync_copy(src4_hbm.at[c, s].at[pl.ds(0, _GC)], gsi.at[0])
    pltpu.sync_copy(dst3_hbm.at[s].at[pl.ds(0, _GC)], gdi.at[0])

    @pl.loop(0, 80)
    def _(r):
        for j in range(dh // 16):
            b3[r, pl.ds(j * 16, 16)] = jnp.zeros((16,), jnp.float32)

    pltpu.async_copy(y_hbm.at[gsi.at[0, 0]], b0, sg0)
    pltpu.async_copy(y_hbm.at[gsi.at[0, 1]], b1, sg1)

    zsrc = b3.at[pl.ds(0, 80)]

    @pl.when(s < 15)
    def _():
        for k in range(8):
            pltpu.sync_copy(zsrc, acc.at[pl.ds(s * 640 + k * 80, 80)])

    @pl.when(s == 15)
    def _():
        for k in range(5):
            pltpu.sync_copy(zsrc, acc.at[pl.ds(9600 + k * 80, 80)])

    plsc.subcore_barrier()

    def gwait(qq, k, buf, sem):
        pltpu.make_async_copy(y_hbm.at[gsi.at[qq, k]], buf, sem).wait()

    def swait(qq, k, buf, sem):
        pltpu.make_async_copy(buf, acc.at[gdi.at[qq, k]], sem).wait()

    @pl.loop(0, _NG // 2)
    def _(gp):
        for qq in range(2):
            g = 2 * gp + qq
            for k in range(_GC):
                a = k % 4
                nxt = (k + 2) % 4

                gwait(qq, k, bufs[a], sgs[a])
                pltpu.async_copy(bufs[a], acc.at[gdi.at[qq, k]], sss[a],
                                 add=True)

                def _wprev():
                    swait(qq, k, bufs[nxt], sss[nxt])

                if k < 2:
                    pl.when(g > 0)(_wprev)
                else:
                    _wprev()

                if k == 1:
                    def _pref():
                        pltpu.sync_copy(
                            src4_hbm.at[c, s].at[pl.ds((g + 1) * _GC, _GC)],
                            gsi.at[1 - qq])
                        pltpu.sync_copy(
                            dst3_hbm.at[s].at[pl.ds((g + 1) * _GC, _GC)],
                            gdi.at[1 - qq])

                    pl.when(g < _NG - 1)(_pref)

                if k < _GC - 2:
                    pltpu.async_copy(y_hbm.at[gsi.at[qq, k + 2]], bufs[nxt],
                                     sgs[nxt])
                else:
                    def _gnext():
                        pltpu.async_copy(
                            y_hbm.at[gsi.at[1 - qq, k - (_GC - 2)]],
                            bufs[nxt], sgs[nxt])

                    pl.when(g < _NG - 1)(_gnext)

    swait(1, _GC - 2, b2, ss2)
    swait(1, _GC - 1, b3, ss3)

    plsc.subcore_barrier()

    stg = b0.at[pl.ds(0, 80)]

    @pl.when(s < 15)
    def _():
        for k in range(8):
            r0 = s * 640 + k * 80
            pltpu.sync_copy(acc.at[pl.ds(r0, 80)], stg)
            pltpu.sync_copy(stg, out_hbm.at[pl.ds(coff + r0, 80)])

    @pl.when(s == 15)
    def _():
        for k in range(5):
            r0 = 9600 + k * 80
            pltpu.sync_copy(acc.at[pl.ds(r0, 80)], stg)
            pltpu.sync_copy(stg, out_hbm.at[pl.ds(coff + r0, 80)])


def _agg(src4, dst3, y2, dh):
    if dh == 128:
        body = functools.partial(_agg_body, dh)
        nbuf, nsem = 2, 3
    else:
        body = functools.partial(_agg_body4, dh)
        nbuf, nsem = 4, 8
    return pl.kernel(
        body,
        out_type=jax.ShapeDtypeStruct((2 * _N, dh), jnp.float32),
        mesh=_sc_mesh(),
        scratch_types=(
            [pltpu.VMEM_SHARED((_N, dh), jnp.float32),
             pltpu.VMEM((2, _GC, _KA), jnp.int32),
             pltpu.VMEM((2, _GC, _KA), jnp.int32)]
            + [pltpu.VMEM((_KA, dh), jnp.float32)] * nbuf
            + [pltpu.SemaphoreType.DMA] * nsem
        ),
        compiler_params=pltpu.CompilerParams(use_tc_tiling_on_sc=False),
    )(src4, dst3, y2)


def _tc1(degp, x, w1):
    def body(degp_ref, x_ref, w_ref, dinv_ref, y_ref):
        deg = degp_ref[0] + degp_ref[1] + 1.0
        dinv = lax.rsqrt(deg)
        xw = jnp.dot(x_ref[...], w_ref[...],
                     preferred_element_type=jnp.float32)
        y = xw * dinv
        dinv_ref[...] = dinv
        y_ref[0] = y[:, :128]
        y_ref[1] = y[:, 128:]

    return pl.pallas_call(
        body,
        grid=(_N // _NB,),
        in_specs=[
            pl.BlockSpec((2, _NB, 1), lambda i: (0, i, 0)),
            pl.BlockSpec((_NB, 512), lambda i: (i, 0)),
            pl.BlockSpec((512, 256), lambda i: (0, 0)),
        ],
        out_specs=[
            pl.BlockSpec((_NB, 1), lambda i: (i, 0)),
            pl.BlockSpec((2, _NB, 128), lambda i: (0, i, 0)),
        ],
        out_shape=[
            jax.ShapeDtypeStruct((_N, 1), jnp.float32),
            jax.ShapeDtypeStruct((2, _N, 128), jnp.float32),
        ],
    )(degp, x, w1)


def _tc_mid(agg, y, dinv, b, w, dh_in, dh_out):
    d_in = 2 * dh_in

    def body(agg_ref, y_ref, dinv_ref, b_ref, w_ref, yout_ref):
        dv = dinv_ref[...]
        bv = b_ref[...]
        h0 = jnp.maximum(dv * (agg_ref[0] + y_ref[0]) + bv[:dh_in], 0.0)
        h1 = jnp.maximum(dv * (agg_ref[1] + y_ref[1]) + bv[dh_in:], 0.0)
        h = jnp.concatenate([h0, h1], axis=1)
        xw = jnp.dot(h, w_ref[...], preferred_element_type=jnp.float32)
        yv = xw * dv
        yout_ref[0] = yv[:, :dh_out]
        yout_ref[1] = yv[:, dh_out:]

    return pl.pallas_call(
        body,
        grid=(_N // _NB,),
        in_specs=[
            pl.BlockSpec((2, _NB, dh_in), lambda i: (0, i, 0)),
            pl.BlockSpec((2, _NB, dh_in), lambda i: (0, i, 0)),
            pl.BlockSpec((_NB, 1), lambda i: (i, 0)),
            pl.BlockSpec((d_in,), lambda i: (0,)),
            pl.BlockSpec((d_in, 2 * dh_out), lambda i: (0, 0)),
        ],
        out_specs=pl.BlockSpec((2, _NB, dh_out), lambda i: (0, i, 0)),
        out_shape=jax.ShapeDtypeStruct((2, _N, dh_out), jnp.float32),
    )(agg, y, dinv, b, w)


def _tc5(agg, y, dinv, b4, batch2, lw1, lb1, lw2, lb2):
    nblk = _N // _NB

    def body(agg_ref, y_ref, dinv_ref, b_ref, batch_ref, lw1_ref, lb1_ref,
             lw2_ref, lb2_ref, out_ref, sums, cnt):
        i = pl.program_id(0)
        dv = dinv_ref[...]
        bv = b_ref[...]
        h0 = dv * (agg_ref[0] + y_ref[0]) + bv[:16]
        h1 = dv * (agg_ref[1] + y_ref[1]) + bv[16:]
        h = jnp.concatenate([h0, h1], axis=1)
        gi = lax.broadcasted_iota(jnp.int32, (_NB, _G), 1)
        oh = (batch_ref[...] == gi).astype(jnp.float32)
        ps = lax.dot_general(oh, h, (((0,), (0,)), ((), ())),
                             preferred_element_type=jnp.float32)
        pc = lax.dot_general(oh, jnp.ones((_NB, 1), jnp.float32),
                             (((0,), (0,)), ((), ())),
                             preferred_element_type=jnp.float32)

        @pl.when(i == 0)
        def _():
            sums[...] = ps
            cnt[...] = pc
            out_ref[...] = jnp.zeros_like(out_ref)

        @pl.when(i > 0)
        def _():
            sums[...] = sums[...] + ps
            cnt[...] = cnt[...] + pc

        @pl.when(i == nblk - 1)
        def _():
            pooled = sums[...] / jnp.maximum(cnt[...], 1.0)
            z = jnp.maximum(
                jnp.dot(pooled, lw1_ref[...],
                        preferred_element_type=jnp.float32) + lb1_ref[...],
                0.0)
            out_ref[...] = jnp.dot(
                z, lw2_ref[...], preferred_element_type=jnp.float32) \
                + lb2_ref[...]

    return pl.pallas_call(
        body,
        grid=(nblk,),
        in_specs=[
            pl.BlockSpec((2, _NB, 16), lambda i: (0, i, 0)),
            pl.BlockSpec((2, _NB, 16), lambda i: (0, i, 0)),
            pl.BlockSpec((_NB, 1), lambda i: (i, 0)),
            pl.BlockSpec((32,), lambda i: (0,)),
            pl.BlockSpec((_NB, 1), lambda i: (i, 0)),
            pl.BlockSpec((32, 16), lambda i: (0, 0)),
            pl.BlockSpec((16,), lambda i: (0,)),
            pl.BlockSpec((16, 2), lambda i: (0, 0)),
            pl.BlockSpec((2,), lambda i: (0,)),
        ],
        out_specs=pl.BlockSpec((_G, 2), lambda i: (0, 0)),
        out_shape=jax.ShapeDtypeStruct((_G, 2), jnp.float32),
        scratch_shapes=[
            pltpu.VMEM((_G, 32), jnp.float32),
            pltpu.VMEM((_G, 1), jnp.float32),
        ],
    )(agg, y, dinv, b4, batch2, lw1, lb1, lw2, lb2)


def kernel(x, edge_index, batch, w1, b1, w2, b2, w3, b3, w4, b4, lw1, lb1,
           lw2, lb2):
    src = edge_index[0]
    dst = edge_index[1]
    src4 = jnp.stack([src, src + _N]).reshape(2, _NS, _NCH, _KA)
    dst3 = dst.reshape(_NS, _NCH, _KA)
    degp = _deg(dst).reshape(2, _N, 1)
    dinv, y1 = _tc1(degp, x, w1)
    agg1 = _agg(src4, dst3, y1.reshape(2 * _N, 128), 128).reshape(2, _N, 128)
    y2 = _tc_mid(agg1, y1, dinv, b1, w2, 128, 64)
    agg2 = _agg(src4, dst3, y2.reshape(2 * _N, 64), 64).reshape(2, _N, 64)
    y3 = _tc_mid(agg2, y2, dinv, b2, w3, 64, 32)
    agg3 = _agg(src4, dst3, y3.reshape(2 * _N, 32), 32).reshape(2, _N, 32)
    y4 = _tc_mid(agg3, y3, dinv, b3, w4, 32, 16)
    agg4 = _agg(src4, dst3, y4.reshape(2 * _N, 16), 16).reshape(2, _N, 16)
    return _tc5(agg4, y4, dinv, b4, batch[:, None], lw1, lb1, lw2, lb2)


# speedup vs baseline: 15.8770x; 1.1139x over previous
import functools

import jax
import jax.numpy as jnp
from jax import lax
from jax.experimental import pallas as pl
from jax.experimental.pallas import tpu as pltpu
from jax.experimental.pallas import tpu_sc as plsc

_N = 10000
_E = 160000
_G = 64
_NB = 1000
_NC = 2
_NS = 16


def _sc_mesh():
    return plsc.VectorSubcoreMesh(
        core_axis_name="c", subcore_axis_name="s", num_cores=_NC,
        num_subcores=_NS)


def _deg_body(dst16_hbm, out_hbm, shr, acc, didx2, blk):
    c = lax.axis_index("c")
    s = lax.axis_index("s")
    ones = jnp.full((16,), 1.0, jnp.float32)

    @pl.when(c == 0)
    def _():
        pltpu.sync_copy(dst16_hbm.at[s].at[pl.ds(0, 313)],
                        didx2.at[pl.ds(0, 313)])

    @pl.when(c == 1)
    def _():
        pltpu.sync_copy(dst16_hbm.at[s].at[pl.ds(313, 312)],
                        didx2.at[pl.ds(0, 312)])

    @pl.loop(0, _N // 16)
    def _(i):
        acc[pl.ds(i * 16, 16)] = jnp.zeros((16,), jnp.float32)

    @pl.when(c == 0)
    def _():
        @pl.loop(0, 313)
        def _(i):
            plsc.addupdate_scatter(acc, [didx2[i]], ones)

    @pl.when(c == 1)
    def _():
        @pl.loop(0, 312)
        def _(i):
            plsc.addupdate_scatter(acc, [didx2[i]], ones)

    pltpu.sync_copy(acc, shr.at[s])
    plsc.subcore_barrier()

    ncol = 640
    col0 = s * 640

    @pl.when(s < 15)
    def _():
        pltpu.sync_copy(shr.at[:, pl.ds(col0, ncol)], blk)

    @pl.when(s == 15)
    def _():
        pltpu.sync_copy(shr.at[:, pl.ds(9600, 400)], blk.at[:, pl.ds(0, 400)])

    @pl.loop(0, 40)
    def _(j):
        v = blk[0, pl.ds(j * 16, 16)]
        for t in range(1, 16):
            v = v + blk[t, pl.ds(j * 16, 16)]
        acc[pl.ds(j * 16, 16)] = v

    @pl.when(s < 15)
    def _():
        pltpu.sync_copy(acc.at[pl.ds(0, 640)],
                        out_hbm.at[pl.ds(c * _N + col0, 640)])

    @pl.when(s == 15)
    def _():
        pltpu.sync_copy(acc.at[pl.ds(0, 400)],
                        out_hbm.at[pl.ds(c * _N + 9600, 400)])


def _deg(dst16):
    return pl.kernel(
        _deg_body,
        out_type=jax.ShapeDtypeStruct((2 * _N,), jnp.float32),
        mesh=_sc_mesh(),
        scratch_types=[
            pltpu.VMEM_SHARED((_NS, _N), jnp.float32),
            pltpu.VMEM((_N,), jnp.float32),
            pltpu.VMEM((313, 16), jnp.int32),
            pltpu.VMEM((_NS, 640), jnp.float32),
        ],
        compiler_params=pltpu.CompilerParams(use_tc_tiling_on_sc=False,
                                             needs_layout_passes=False),
    )(dst16)


_KA = 125
_NCH = 80


_GC = 8
_NG = _NCH // _GC


def _agg_body(dh, src4_hbm, dst3_hbm, y_hbm, out_hbm, acc, gsi, gdi,
              b0, b1, sg, ss0, ss1):
    c = lax.axis_index("c")
    s = lax.axis_index("s")
    coff = c * _N

    bufs = (b0, b1)
    sss = (ss0, ss1)

    pltpu.sync_copy(src4_hbm.at[c, s].at[pl.ds(0, _GC)], gsi.at[0])
    pltpu.sync_copy(dst3_hbm.at[s].at[pl.ds(0, _GC)], gdi.at[0])

    @pl.loop(0, 80)
    def _(r):
        for j in range(dh // 16):
            b1[r, pl.ds(j * 16, 16)] = jnp.zeros((16,), jnp.float32)

    pltpu.async_copy(y_hbm.at[gsi.at[0, 0]], b0, sg)

    zsrc = b1.at[pl.ds(0, 80)]

    @pl.when(s < 15)
    def _():
        for k in range(8):
            pltpu.sync_copy(zsrc, acc.at[pl.ds(s * 640 + k * 80, 80)])

    @pl.when(s == 15)
    def _():
        for k in range(5):
            pltpu.sync_copy(zsrc, acc.at[pl.ds(9600 + k * 80, 80)])

    plsc.subcore_barrier()

    def gwait(qq, k, buf):
        pltpu.make_async_copy(y_hbm.at[gsi.at[qq, k]], buf, sg).wait()

    def swait(qq, k, buf, sem):
        pltpu.make_async_copy(buf, acc.at[gdi.at[qq, k]], sem).wait()

    @pl.loop(0, _NG // 2)
    def _(gp):
        for qq in range(2):
            g = 2 * gp + qq
            for k in range(_GC):
                x = bufs[k % 2]
                o = bufs[1 - k % 2]
                gwait(qq, k, x)
                pltpu.async_copy(x, acc.at[gdi.at[qq, k]], sss[k % 2],
                                 add=True)
                def _wprev():
                    swait(qq, k, o, sss[1 - k % 2])

                if k == 0:
                    pl.when(g > 0)(_wprev)
                else:
                    _wprev()

                if k == 1:
                    def _pref():
                        pltpu.sync_copy(
                            src4_hbm.at[c, s].at[pl.ds((g + 1) * _GC, _GC)],
                            gsi.at[1 - qq])
                        pltpu.sync_copy(
                            dst3_hbm.at[s].at[pl.ds((g + 1) * _GC, _GC)],
                            gdi.at[1 - qq])

                    pl.when(g < _NG - 1)(_pref)

                if k < _GC - 1:
                    pltpu.async_copy(y_hbm.at[gsi.at[qq, k + 1]], o, sg)
                else:
                    def _gnext():
                        pltpu.async_copy(y_hbm.at[gsi.at[1 - qq, 0]], o, sg)

                    pl.when(g < _NG - 1)(_gnext)

    swait(1, _GC - 1, b1, ss1)

    plsc.subcore_barrier()

    stg = b0.at[pl.ds(0, 80)]

    @pl.when(s < 15)
    def _():
        for k in range(8):
            r0 = s * 640 + k * 80
            pltpu.sync_copy(acc.at[pl.ds(r0, 80)], stg)
            pltpu.sync_copy(stg, out_hbm.at[pl.ds(coff + r0, 80)])

    @pl.when(s == 15)
    def _():
        for k in range(5):
            r0 = 9600 + k * 80
            pltpu.sync_copy(acc.at[pl.ds(r0, 80)], stg)
            pltpu.sync_copy(stg, out_hbm.at[pl.ds(coff + r0, 80)])


def _agg_body4(dh, src4_hbm, dst3_hbm, y_hbm, out_hbm, acc, gsi, gdi,
               b0, b1, b2, b3, sg0, sg1, sg2, sg3, ss0, ss1, ss2, ss3):
    c = lax.axis_index("c")
    s = lax.axis_index("s")
    coff = c * _N

    bufs = (b0, b1, b2, b3)
    sgs = (sg0, sg1, sg2, sg3)
    sss = (ss0, ss1, ss2, ss3)

    pltpu.sync_copy(src4_hbm.at[c, s].at[pl.ds(0, _GC)], gsi.at[0])
    pltpu.sync_copy(dst3_hbm.at[s].at[pl.ds(0, _GC)], gdi.at[0])

    @pl.loop(0, 80)
    def _(r):
        for j in range(dh // 16):
            b3[r, pl.ds(j * 16, 16)] = jnp.zeros((16,), jnp.float32)

    pltpu.async_copy(y_hbm.at[gsi.at[0, 0]], b0, sg0)
    pltpu.async_copy(y_hbm.at[gsi.at[0, 1]], b1, sg1)

    zsrc = b3.at[pl.ds(0, 80)]

    @pl.when(s < 15)
    def _():
        for k in range(8):
            pltpu.sync_copy(zsrc, acc.at[pl.ds(s * 640 + k * 80, 80)])

    @pl.when(s == 15)
    def _():
        for k in range(5):
            pltpu.sync_copy(zsrc, acc.at[pl.ds(9600 + k * 80, 80)])

    plsc.subcore_barrier()

    def gwait(qq, k, buf, sem):
        pltpu.make_async_copy(y_hbm.at[gsi.at[qq, k]], buf, sem).wait()

    def swait(qq, k, buf, sem):
        pltpu.make_async_copy(buf, acc.at[gdi.at[qq, k]], sem).wait()

    @pl.loop(0, _NG // 2)
    def _(gp):
        for qq in range(2):
            g = 2 * gp + qq
            for k in range(_GC):
                a = k % 4
                nxt = (k + 2) % 4

                gwait(qq, k, bufs[a], sgs[a])
                pltpu.async_copy(bufs[a], acc.at[gdi.at[qq, k]], sss[a],
                                 add=True)

                def _wprev():
                    swait(qq, k, bufs[nxt], sss[nxt])

                if k < 2:
                    pl.when(g > 0)(_wprev)
                else:
                    _wprev()

                if k == 1:
                    def _pref():
                        pltpu.sync_copy(
                            src4_hbm.at[c, s].at[pl.ds((g + 1) * _GC, _GC)],
                            gsi.at[1 - qq])
                        pltpu.sync_copy(
                            dst3_hbm.at[s].at[pl.ds((g + 1) * _GC, _GC)],
                            gdi.at[1 - qq])

                    pl.when(g < _NG - 1)(_pref)

                if k < _GC - 2:
                    pltpu.async_copy(y_hbm.at[gsi.at[qq, k + 2]], bufs[nxt],
                                     sgs[nxt])
                else:
                    def _gnext():
                        pltpu.async_copy(
                            y_hbm.at[gsi.at[1 - qq, k - (_GC - 2)]],
                            bufs[nxt], sgs[nxt])

                    pl.when(g < _NG - 1)(_gnext)

    swait(1, _GC - 2, b2, ss2)
    swait(1, _GC - 1, b3, ss3)

    plsc.subcore_barrier()

    stg = b0.at[pl.ds(0, 80)]

    @pl.when(s < 15)
    def _():
        for k in range(8):
            r0 = s * 640 + k * 80
            pltpu.sync_copy(acc.at[pl.ds(r0, 80)], stg)
            pltpu.sync_copy(stg, out_hbm.at[pl.ds(coff + r0, 80)])

    @pl.when(s == 15)
    def _():
        for k in range(5):
            r0 = 9600 + k * 80
            pltpu.sync_copy(acc.at[pl.ds(r0, 80)], stg)
            pltpu.sync_copy(stg, out_hbm.at[pl.ds(coff + r0, 80)])


def _agg(src4, dst3, y2, dh):
    if dh == 128:
        body = functools.partial(_agg_body, dh)
        nbuf, nsem = 2, 3
    else:
        body = functools.partial(_agg_body4, dh)
        nbuf, nsem = 4, 8
    return pl.kernel(
        body,
        out_type=jax.ShapeDtypeStruct((2 * _N, dh), jnp.float32),
        mesh=_sc_mesh(),
        scratch_types=(
            [pltpu.VMEM_SHARED((_N, dh), jnp.float32),
             pltpu.VMEM((2, _GC, _KA), jnp.int32),
             pltpu.VMEM((2, _GC, _KA), jnp.int32)]
            + [pltpu.VMEM((_KA, dh), jnp.float32)] * nbuf
            + [pltpu.SemaphoreType.DMA] * nsem
        ),
        compiler_params=pltpu.CompilerParams(use_tc_tiling_on_sc=False),
    )(src4, dst3, y2)


def _tc1(degp, x, w1):
    def body(degp_ref, x_ref, w_ref, dinv_ref, y_ref):
        deg = degp_ref[0] + degp_ref[1] + 1.0
        dinv = lax.rsqrt(deg)
        xw = jnp.dot(x_ref[...], w_ref[...],
                     preferred_element_type=jnp.float32)
        y = xw * dinv
        dinv_ref[...] = dinv
        y_ref[0] = y[:, :128]
        y_ref[1] = y[:, 128:]

    return pl.pallas_call(
        body,
        grid=(_N // _NB,),
        in_specs=[
            pl.BlockSpec((2, _NB, 1), lambda i: (0, i, 0)),
            pl.BlockSpec((_NB, 512), lambda i: (i, 0)),
            pl.BlockSpec((512, 256), lambda i: (0, 0)),
        ],
        out_specs=[
            pl.BlockSpec((_NB, 1), lambda i: (i, 0)),
            pl.BlockSpec((2, _NB, 128), lambda i: (0, i, 0)),
        ],
        out_shape=[
            jax.ShapeDtypeStruct((_N, 1), jnp.float32),
            jax.ShapeDtypeStruct((2, _N, 128), jnp.float32),
        ],
    )(degp, x, w1)


def _tc_mid(agg, y, dinv, b, w, dh_in, dh_out):
    d_in = 2 * dh_in

    def body(agg_ref, y_ref, dinv_ref, b_ref, w_ref, yout_ref):
        dv = dinv_ref[...]
        bv = b_ref[...]
        h0 = jnp.maximum(dv * (agg_ref[0] + y_ref[0]) + bv[:dh_in], 0.0)
        h1 = jnp.maximum(dv * (agg_ref[1] + y_ref[1]) + bv[dh_in:], 0.0)
        h = jnp.concatenate([h0, h1], axis=1)
        xw = jnp.dot(h, w_ref[...], preferred_element_type=jnp.float32)
        yv = xw * dv
        yout_ref[0] = yv[:, :dh_out]
        yout_ref[1] = yv[:, dh_out:]

    return pl.pallas_call(
        body,
        grid=(_N // _NB,),
        in_specs=[
            pl.BlockSpec((2, _NB, dh_in), lambda i: (0, i, 0)),
            pl.BlockSpec((2, _NB, dh_in), lambda i: (0, i, 0)),
            pl.BlockSpec((_NB, 1), lambda i: (i, 0)),
            pl.BlockSpec((d_in,), lambda i: (0,)),
            pl.BlockSpec((d_in, 2 * dh_out), lambda i: (0, 0)),
        ],
        out_specs=pl.BlockSpec((2, _NB, dh_out), lambda i: (0, i, 0)),
        out_shape=jax.ShapeDtypeStruct((2, _N, dh_out), jnp.float32),
    )(agg, y, dinv, b, w)


def _tc5(agg, y, dinv, b4, batch2, lw1, lb1, lw2, lb2):
    nblk = _N // _NB

    def body(agg_ref, y_ref, dinv_ref, b_ref, batch_ref, lw1_ref, lb1_ref,
             lw2_ref, lb2_ref, out_ref, sums, cnt):
        i = pl.program_id(0)
        dv = dinv_ref[...]
        bv = b_ref[...]
        h0 = dv * (agg_ref[0] + y_ref[0]) + bv[:16]
        h1 = dv * (agg_ref[1] + y_ref[1]) + bv[16:]
        h = jnp.concatenate([h0, h1], axis=1)
        gi = lax.broadcasted_iota(jnp.int32, (_NB, _G), 1)
        oh = (batch_ref[...] == gi).astype(jnp.float32)
        ps = lax.dot_general(oh, h, (((0,), (0,)), ((), ())),
                             preferred_element_type=jnp.float32)
        pc = lax.dot_general(oh, jnp.ones((_NB, 1), jnp.float32),
                             (((0,), (0,)), ((), ())),
                             preferred_element_type=jnp.float32)

        @pl.when(i == 0)
        def _():
            sums[...] = ps
            cnt[...] = pc
            out_ref[...] = jnp.zeros_like(out_ref)

        @pl.when(i > 0)
        def _():
            sums[...] = sums[...] + ps
            cnt[...] = cnt[...] + pc

        @pl.when(i == nblk - 1)
        def _():
            pooled = sums[...] / jnp.maximum(cnt[...], 1.0)
            z = jnp.maximum(
                jnp.dot(pooled, lw1_ref[...],
                        preferred_element_type=jnp.float32) + lb1_ref[...],
                0.0)
            out_ref[...] = jnp.dot(
                z, lw2_ref[...], preferred_element_type=jnp.float32) \
                + lb2_ref[...]

    return pl.pallas_call(
        body,
        grid=(nblk,),
        in_specs=[
            pl.BlockSpec((2, _NB, 16), lambda i: (0, i, 0)),
            pl.BlockSpec((2, _NB, 16), lambda i: (0, i, 0)),
            pl.BlockSpec((_NB, 1), lambda i: (i, 0)),
            pl.BlockSpec((32,), lambda i: (0,)),
            pl.BlockSpec((_NB, 1), lambda i: (i, 0)),
            pl.BlockSpec((32, 16), lambda i: (0, 0)),
            pl.BlockSpec((16,), lambda i: (0,)),
            pl.BlockSpec((16, 2), lambda i: (0, 0)),
            pl.BlockSpec((2,), lambda i: (0,)),
        ],
        out_specs=pl.BlockSpec((_G, 2), lambda i: (0, 0)),
        out_shape=jax.ShapeDtypeStruct((_G, 2), jnp.float32),
        scratch_shapes=[
            pltpu.VMEM((_G, 32), jnp.float32),
            pltpu.VMEM((_G, 1), jnp.float32),
        ],
    )(agg, y, dinv, b4, batch2, lw1, lb1, lw2, lb2)


def kernel(x, edge_index, batch, w1, b1, w2, b2, w3, b3, w4, b4, lw1, lb1,
           lw2, lb2):
    src = edge_index[0]
    dst = edge_index[1]
    src4 = jnp.stack([src, src + _N]).reshape(2, _NS, _NCH, _KA)
    dst3 = dst.reshape(_NS, _NCH, _KA)
    degp = _deg(dst.reshape(_NS, _N // 16, 16)).reshape(2, _N, 1)
    dinv, y1 = _tc1(degp, x, w1)
    agg1 = _agg(src4, dst3, y1.reshape(2 * _N, 128), 128).reshape(2, _N, 128)
    y2 = _tc_mid(agg1, y1, dinv, b1, w2, 128, 64)
    agg2 = _agg(src4, dst3, y2.reshape(2 * _N, 64), 64).reshape(2, _N, 64)
    y3 = _tc_mid(agg2, y2, dinv, b2, w3, 64, 32)
    agg3 = _agg(src4, dst3, y3.reshape(2 * _N, 32), 32).reshape(2, _N, 32)
    y4 = _tc_mid(agg3, y3, dinv, b3, w4, 32, 16)
    agg4 = _agg(src4, dst3, y4.reshape(2 * _N, 16), 16).reshape(2, _N, 16)
    return _tc5(agg4, y4, dinv, b4, batch[:, None], lw1, lb1, lw2, lb2)


# speedup vs baseline: 16.7282x; 1.0536x over previous
import functools

import jax
import jax.numpy as jnp
from jax import lax
from jax.experimental import pallas as pl
from jax.experimental.pallas import tpu as pltpu
from jax.experimental.pallas import tpu_sc as plsc

_N = 10000
_E = 160000
_G = 64
_NB = 1000
_NC = 2
_NS = 16


def _sc_mesh():
    return plsc.VectorSubcoreMesh(
        core_axis_name="c", subcore_axis_name="s", num_cores=_NC,
        num_subcores=_NS)


def _deg_body(dst16_hbm, out_hbm, shr, acc, didx2, blk):
    c = lax.axis_index("c")
    s = lax.axis_index("s")
    ones = jnp.full((16,), 1.0, jnp.float32)

    @pl.when(c == 0)
    def _():
        pltpu.sync_copy(dst16_hbm.at[s].at[pl.ds(0, 313)],
                        didx2.at[pl.ds(0, 313)])

    @pl.when(c == 1)
    def _():
        pltpu.sync_copy(dst16_hbm.at[s].at[pl.ds(313, 312)],
                        didx2.at[pl.ds(0, 312)])

    @pl.loop(0, _N // 16)
    def _(i):
        acc[pl.ds(i * 16, 16)] = jnp.zeros((16,), jnp.float32)

    @pl.when(c == 0)
    def _():
        @pl.loop(0, 313)
        def _(i):
            plsc.addupdate_scatter(acc, [didx2[i]], ones)

    @pl.when(c == 1)
    def _():
        @pl.loop(0, 312)
        def _(i):
            plsc.addupdate_scatter(acc, [didx2[i]], ones)

    pltpu.sync_copy(acc, shr.at[s])
    plsc.subcore_barrier()

    ncol = 640
    col0 = s * 640

    @pl.when(s < 15)
    def _():
        pltpu.sync_copy(shr.at[:, pl.ds(col0, ncol)], blk)

    @pl.when(s == 15)
    def _():
        pltpu.sync_copy(shr.at[:, pl.ds(9600, 400)], blk.at[:, pl.ds(0, 400)])

    @pl.loop(0, 40)
    def _(j):
        v = blk[0, pl.ds(j * 16, 16)]
        for t in range(1, 16):
            v = v + blk[t, pl.ds(j * 16, 16)]
        acc[pl.ds(j * 16, 16)] = v

    @pl.when(s < 15)
    def _():
        pltpu.sync_copy(acc.at[pl.ds(0, 640)],
                        out_hbm.at[pl.ds(c * _N + col0, 640)])

    @pl.when(s == 15)
    def _():
        pltpu.sync_copy(acc.at[pl.ds(0, 400)],
                        out_hbm.at[pl.ds(c * _N + 9600, 400)])


def _deg(dst16):
    return pl.kernel(
        _deg_body,
        out_type=jax.ShapeDtypeStruct((2 * _N,), jnp.float32),
        mesh=_sc_mesh(),
        scratch_types=[
            pltpu.VMEM_SHARED((_NS, _N), jnp.float32),
            pltpu.VMEM((_N,), jnp.float32),
            pltpu.VMEM((313, 16), jnp.int32),
            pltpu.VMEM((_NS, 640), jnp.float32),
        ],
        compiler_params=pltpu.CompilerParams(use_tc_tiling_on_sc=False,
                                             needs_layout_passes=False),
    )(dst16)


_KA = 125
_NCH = 80


_GC = 8
_NG = _NCH // _GC


def _agg_body4(dh, dt, src4_hbm, dst3_hbm, y_hbm, out_hbm, acc, gsi, gdi,
               b0, b1, b2, b3, sg0, sg1, sg2, sg3, ss0, ss1, ss2, ss3):
    c = lax.axis_index("c")
    s = lax.axis_index("s")
    coff = c * _N

    bufs = (b0, b1, b2, b3)
    sgs = (sg0, sg1, sg2, sg3)
    sss = (ss0, ss1, ss2, ss3)

    pltpu.sync_copy(src4_hbm.at[c, s].at[pl.ds(0, _GC)], gsi.at[0])
    pltpu.sync_copy(dst3_hbm.at[s].at[pl.ds(0, _GC)], gdi.at[0])

    vw = 16 if dt == jnp.float32 else 32

    @pl.loop(0, 80)
    def _(r):
        for j in range(dh // vw):
            b3[r, pl.ds(j * vw, vw)] = jnp.zeros((vw,), dt)

    pltpu.async_copy(y_hbm.at[gsi.at[0, 0]], b0, sg0)
    pltpu.async_copy(y_hbm.at[gsi.at[0, 1]], b1, sg1)

    zsrc = b3.at[pl.ds(0, 80)]

    @pl.when(s < 15)
    def _():
        for k in range(8):
            pltpu.sync_copy(zsrc, acc.at[pl.ds(s * 640 + k * 80, 80)])

    @pl.when(s == 15)
    def _():
        for k in range(5):
            pltpu.sync_copy(zsrc, acc.at[pl.ds(9600 + k * 80, 80)])

    plsc.subcore_barrier()

    def gwait(qq, k, buf, sem):
        pltpu.make_async_copy(y_hbm.at[gsi.at[qq, k]], buf, sem).wait()

    def swait(qq, k, buf, sem):
        pltpu.make_async_copy(buf, acc.at[gdi.at[qq, k]], sem).wait()

    @pl.loop(0, _NG // 2)
    def _(gp):
        for qq in range(2):
            g = 2 * gp + qq
            for k in range(_GC):
                a = k % 4
                nxt = (k + 2) % 4

                gwait(qq, k, bufs[a], sgs[a])
                pltpu.async_copy(bufs[a], acc.at[gdi.at[qq, k]], sss[a],
                                 add=True)

                def _wprev():
                    swait(qq, k, bufs[nxt], sss[nxt])

                if k < 2:
                    pl.when(g > 0)(_wprev)
                else:
                    _wprev()

                if k == 1:
                    def _pref():
                        pltpu.sync_copy(
                            src4_hbm.at[c, s].at[pl.ds((g + 1) * _GC, _GC)],
                            gsi.at[1 - qq])
                        pltpu.sync_copy(
                            dst3_hbm.at[s].at[pl.ds((g + 1) * _GC, _GC)],
                            gdi.at[1 - qq])

                    pl.when(g < _NG - 1)(_pref)

                if k < _GC - 2:
                    pltpu.async_copy(y_hbm.at[gsi.at[qq, k + 2]], bufs[nxt],
                                     sgs[nxt])
                else:
                    def _gnext():
                        pltpu.async_copy(
                            y_hbm.at[gsi.at[1 - qq, k - (_GC - 2)]],
                            bufs[nxt], sgs[nxt])

                    pl.when(g < _NG - 1)(_gnext)

    swait(1, _GC - 2, b2, ss2)
    swait(1, _GC - 1, b3, ss3)

    plsc.subcore_barrier()

    stg = b0.at[pl.ds(0, 80)]

    @pl.when(s < 15)
    def _():
        for k in range(8):
            r0 = s * 640 + k * 80
            pltpu.sync_copy(acc.at[pl.ds(r0, 80)], stg)
            pltpu.sync_copy(stg, out_hbm.at[pl.ds(coff + r0, 80)])

    @pl.when(s == 15)
    def _():
        for k in range(5):
            r0 = 9600 + k * 80
            pltpu.sync_copy(acc.at[pl.ds(r0, 80)], stg)
            pltpu.sync_copy(stg, out_hbm.at[pl.ds(coff + r0, 80)])


def _agg(src4, dst3, y2, dh):
    dt = y2.dtype
    body = functools.partial(_agg_body4, dh, dt)
    return pl.kernel(
        body,
        out_type=jax.ShapeDtypeStruct((2 * _N, dh), dt),
        mesh=_sc_mesh(),
        scratch_types=(
            [pltpu.VMEM_SHARED((_N, dh), dt),
             pltpu.VMEM((2, _GC, _KA), jnp.int32),
             pltpu.VMEM((2, _GC, _KA), jnp.int32)]
            + [pltpu.VMEM((_KA, dh), dt)] * 4
            + [pltpu.SemaphoreType.DMA] * 8
        ),
        compiler_params=pltpu.CompilerParams(use_tc_tiling_on_sc=False),
    )(src4, dst3, y2)


def _tc1(degp, x, w1):
    def body(degp_ref, x_ref, w_ref, dinv_ref, y_ref):
        deg = degp_ref[0] + degp_ref[1] + 1.0
        dinv = lax.rsqrt(deg)
        xw = jnp.dot(x_ref[...], w_ref[...],
                     preferred_element_type=jnp.float32)
        y = (xw * dinv).astype(jnp.bfloat16)
        dinv_ref[...] = dinv
        y_ref[0] = y[:, :128]
        y_ref[1] = y[:, 128:]

    return pl.pallas_call(
        body,
        grid=(_N // _NB,),
        in_specs=[
            pl.BlockSpec((2, _NB, 1), lambda i: (0, i, 0)),
            pl.BlockSpec((_NB, 512), lambda i: (i, 0)),
            pl.BlockSpec((512, 256), lambda i: (0, 0)),
        ],
        out_specs=[
            pl.BlockSpec((_NB, 1), lambda i: (i, 0)),
            pl.BlockSpec((2, _NB, 128), lambda i: (0, i, 0)),
        ],
        out_shape=[
            jax.ShapeDtypeStruct((_N, 1), jnp.float32),
            jax.ShapeDtypeStruct((2, _N, 128), jnp.bfloat16),
        ],
    )(degp, x, w1)


def _tc_mid(agg, y, dinv, b, w, dh_in, dh_out):
    d_in = 2 * dh_in

    def body(agg_ref, y_ref, dinv_ref, b_ref, w_ref, yout_ref):
        dv = dinv_ref[...]
        bv = b_ref[...]
        a0 = agg_ref[0].astype(jnp.float32) + y_ref[0].astype(jnp.float32)
        a1 = agg_ref[1].astype(jnp.float32) + y_ref[1].astype(jnp.float32)
        h0 = jnp.maximum(dv * a0 + bv[:dh_in], 0.0)
        h1 = jnp.maximum(dv * a1 + bv[dh_in:], 0.0)
        h = jnp.concatenate([h0, h1], axis=1)
        xw = jnp.dot(h, w_ref[...], preferred_element_type=jnp.float32)
        yv = xw * dv
        yout_ref[0] = yv[:, :dh_out]
        yout_ref[1] = yv[:, dh_out:]

    return pl.pallas_call(
        body,
        grid=(_N // _NB,),
        in_specs=[
            pl.BlockSpec((2, _NB, dh_in), lambda i: (0, i, 0)),
            pl.BlockSpec((2, _NB, dh_in), lambda i: (0, i, 0)),
            pl.BlockSpec((_NB, 1), lambda i: (i, 0)),
            pl.BlockSpec((d_in,), lambda i: (0,)),
            pl.BlockSpec((d_in, 2 * dh_out), lambda i: (0, 0)),
        ],
        out_specs=pl.BlockSpec((2, _NB, dh_out), lambda i: (0, i, 0)),
        out_shape=jax.ShapeDtypeStruct((2, _N, dh_out), jnp.float32),
    )(agg, y, dinv, b, w)


def _tc5(agg, y, dinv, b4, batch2, lw1, lb1, lw2, lb2):
    nblk = _N // _NB

    def body(agg_ref, y_ref, dinv_ref, b_ref, batch_ref, lw1_ref, lb1_ref,
             lw2_ref, lb2_ref, out_ref, sums, cnt):
        i = pl.program_id(0)
        dv = dinv_ref[...]
        bv = b_ref[...]
        h0 = dv * (agg_ref[0] + y_ref[0]) + bv[:16]
        h1 = dv * (agg_ref[1] + y_ref[1]) + bv[16:]
        h = jnp.concatenate([h0, h1], axis=1)
        gi = lax.broadcasted_iota(jnp.int32, (_NB, _G), 1)
        oh = (batch_ref[...] == gi).astype(jnp.float32)
        ps = lax.dot_general(oh, h, (((0,), (0,)), ((), ())),
                             preferred_element_type=jnp.float32)
        pc = lax.dot_general(oh, jnp.ones((_NB, 1), jnp.float32),
                             (((0,), (0,)), ((), ())),
                             preferred_element_type=jnp.float32)

        @pl.when(i == 0)
        def _():
            sums[...] = ps
            cnt[...] = pc
            out_ref[...] = jnp.zeros_like(out_ref)

        @pl.when(i > 0)
        def _():
            sums[...] = sums[...] + ps
            cnt[...] = cnt[...] + pc

        @pl.when(i == nblk - 1)
        def _():
            pooled = sums[...] / jnp.maximum(cnt[...], 1.0)
            z = jnp.maximum(
                jnp.dot(pooled, lw1_ref[...],
                        preferred_element_type=jnp.float32) + lb1_ref[...],
                0.0)
            out_ref[...] = jnp.dot(
                z, lw2_ref[...], preferred_element_type=jnp.float32) \
                + lb2_ref[...]

    return pl.pallas_call(
        body,
        grid=(nblk,),
        in_specs=[
            pl.BlockSpec((2, _NB, 16), lambda i: (0, i, 0)),
            pl.BlockSpec((2, _NB, 16), lambda i: (0, i, 0)),
            pl.BlockSpec((_NB, 1), lambda i: (i, 0)),
            pl.BlockSpec((32,), lambda i: (0,)),
            pl.BlockSpec((_NB, 1), lambda i: (i, 0)),
            pl.BlockSpec((32, 16), lambda i: (0, 0)),
            pl.BlockSpec((16,), lambda i: (0,)),
            pl.BlockSpec((16, 2), lambda i: (0, 0)),
            pl.BlockSpec((2,), lambda i: (0,)),
        ],
        out_specs=pl.BlockSpec((_G, 2), lambda i: (0, 0)),
        out_shape=jax.ShapeDtypeStruct((_G, 2), jnp.float32),
        scratch_shapes=[
            pltpu.VMEM((_G, 32), jnp.float32),
            pltpu.VMEM((_G, 1), jnp.float32),
        ],
    )(agg, y, dinv, b4, batch2, lw1, lb1, lw2, lb2)


def kernel(x, edge_index, batch, w1, b1, w2, b2, w3, b3, w4, b4, lw1, lb1,
           lw2, lb2):
    src = edge_index[0]
    dst = edge_index[1]
    src4 = jnp.stack([src, src + _N]).reshape(2, _NS, _NCH, _KA)
    dst3 = dst.reshape(_NS, _NCH, _KA)
    degp = _deg(dst.reshape(_NS, _N // 16, 16)).reshape(2, _N, 1)
    dinv, y1 = _tc1(degp, x, w1)
    agg1 = _agg(src4, dst3, y1.reshape(2 * _N, 128), 128).reshape(2, _N, 128)
    y2 = _tc_mid(agg1, y1, dinv, b1, w2, 128, 64)
    agg2 = _agg(src4, dst3, y2.reshape(2 * _N, 64), 64).reshape(2, _N, 64)
    y3 = _tc_mid(agg2, y2, dinv, b2, w3, 64, 32)
    agg3 = _agg(src4, dst3, y3.reshape(2 * _N, 32), 32).reshape(2, _N, 32)
    y4 = _tc_mid(agg3, y3, dinv, b3, w4, 32, 16)
    agg4 = _agg(src4, dst3, y4.reshape(2 * _N, 16), 16).reshape(2, _N, 16)
    return _tc5(agg4, y4, dinv, b4, batch[:, None], lw1, lb1, lw2, lb2)


# speedup vs baseline: 17.4762x; 1.0447x over previous
import functools

import jax
import jax.numpy as jnp
from jax import lax
from jax.experimental import pallas as pl
from jax.experimental.pallas import tpu as pltpu
from jax.experimental.pallas import tpu_sc as plsc

_N = 10000
_E = 160000
_G = 64
_NB = 1000
_NC = 2
_NS = 16


def _sc_mesh():
    return plsc.VectorSubcoreMesh(
        core_axis_name="c", subcore_axis_name="s", num_cores=_NC,
        num_subcores=_NS)


def _deg_body(dst16_hbm, out_hbm, shr, acc, didx2, blk):
    c = lax.axis_index("c")
    s = lax.axis_index("s")
    ones = jnp.full((16,), 1.0, jnp.float32)

    @pl.when(c == 0)
    def _():
        pltpu.sync_copy(dst16_hbm.at[s].at[pl.ds(0, 313)],
                        didx2.at[pl.ds(0, 313)])

    @pl.when(c == 1)
    def _():
        pltpu.sync_copy(dst16_hbm.at[s].at[pl.ds(313, 312)],
                        didx2.at[pl.ds(0, 312)])

    @pl.loop(0, _N // 16)
    def _(i):
        acc[pl.ds(i * 16, 16)] = jnp.zeros((16,), jnp.float32)

    @pl.when(c == 0)
    def _():
        @pl.loop(0, 313)
        def _(i):
            plsc.addupdate_scatter(acc, [didx2[i]], ones)

    @pl.when(c == 1)
    def _():
        @pl.loop(0, 312)
        def _(i):
            plsc.addupdate_scatter(acc, [didx2[i]], ones)

    pltpu.sync_copy(acc, shr.at[s])
    plsc.subcore_barrier()

    ncol = 640
    col0 = s * 640

    @pl.when(s < 15)
    def _():
        pltpu.sync_copy(shr.at[:, pl.ds(col0, ncol)], blk)

    @pl.when(s == 15)
    def _():
        pltpu.sync_copy(shr.at[:, pl.ds(9600, 400)], blk.at[:, pl.ds(0, 400)])

    @pl.loop(0, 40)
    def _(j):
        v = blk[0, pl.ds(j * 16, 16)]
        for t in range(1, 16):
            v = v + blk[t, pl.ds(j * 16, 16)]
        acc[pl.ds(j * 16, 16)] = v

    @pl.when(s < 15)
    def _():
        pltpu.sync_copy(acc.at[pl.ds(0, 640)],
                        out_hbm.at[pl.ds(c * _N + col0, 640)])

    @pl.when(s == 15)
    def _():
        pltpu.sync_copy(acc.at[pl.ds(0, 400)],
                        out_hbm.at[pl.ds(c * _N + 9600, 400)])


def _deg(dst16):
    return pl.kernel(
        _deg_body,
        out_type=jax.ShapeDtypeStruct((2 * _N,), jnp.float32),
        mesh=_sc_mesh(),
        scratch_types=[
            pltpu.VMEM_SHARED((_NS, _N), jnp.float32),
            pltpu.VMEM((_N,), jnp.float32),
            pltpu.VMEM((313, 16), jnp.int32),
            pltpu.VMEM((_NS, 640), jnp.float32),
        ],
        compiler_params=pltpu.CompilerParams(use_tc_tiling_on_sc=False,
                                             needs_layout_passes=False),
    )(dst16)


_KA = 125
_NCH = 80


_GC = 8
_NG = _NCH // _GC


def _agg_body4(dh, dt, src4_hbm, dst3_hbm, y_hbm, out_hbm, acc, gsi, gdi,
               b0, b1, b2, b3, sg0, sg1, sg2, sg3, ss0, ss1, ss2, ss3):
    c = lax.axis_index("c")
    s = lax.axis_index("s")
    coff = c * _N

    bufs = (b0, b1, b2, b3)
    sgs = (sg0, sg1, sg2, sg3)
    sss = (ss0, ss1, ss2, ss3)

    pltpu.sync_copy(src4_hbm.at[c, s].at[pl.ds(0, _GC)], gsi.at[0])
    pltpu.sync_copy(dst3_hbm.at[s].at[pl.ds(0, _GC)], gdi.at[0])

    vw = 16 if dt == jnp.float32 else 32

    @pl.loop(0, 80)
    def _(r):
        for j in range(dh // vw):
            b3[r, pl.ds(j * vw, vw)] = jnp.zeros((vw,), dt)

    pltpu.async_copy(y_hbm.at[gsi.at[0, 0]], b0, sg0)
    pltpu.async_copy(y_hbm.at[gsi.at[0, 1]], b1, sg1)

    zsrc = b3.at[pl.ds(0, 80)]

    @pl.when(s < 15)
    def _():
        for k in range(8):
            pltpu.sync_copy(zsrc, acc.at[pl.ds(s * 640 + k * 80, 80)])

    @pl.when(s == 15)
    def _():
        for k in range(5):
            pltpu.sync_copy(zsrc, acc.at[pl.ds(9600 + k * 80, 80)])

    plsc.subcore_barrier()

    def gwait(qq, k, buf, sem):
        pltpu.make_async_copy(y_hbm.at[gsi.at[qq, k]], buf, sem).wait()

    def swait(qq, k, buf, sem):
        pltpu.make_async_copy(buf, acc.at[gdi.at[qq, k]], sem).wait()

    @pl.loop(0, _NG // 2)
    def _(gp):
        for qq in range(2):
            g = 2 * gp + qq
            for k in range(_GC):
                a = k % 4
                nxt = (k + 2) % 4

                gwait(qq, k, bufs[a], sgs[a])
                pltpu.async_copy(bufs[a], acc.at[gdi.at[qq, k]], sss[a],
                                 add=True)

                def _wprev():
                    swait(qq, k, bufs[nxt], sss[nxt])

                if k < 2:
                    pl.when(g > 0)(_wprev)
                else:
                    _wprev()

                if k == 1:
                    def _pref():
                        pltpu.sync_copy(
                            src4_hbm.at[c, s].at[pl.ds((g + 1) * _GC, _GC)],
                            gsi.at[1 - qq])
                        pltpu.sync_copy(
                            dst3_hbm.at[s].at[pl.ds((g + 1) * _GC, _GC)],
                            gdi.at[1 - qq])

                    pl.when(g < _NG - 1)(_pref)

                if k < _GC - 2:
                    pltpu.async_copy(y_hbm.at[gsi.at[qq, k + 2]], bufs[nxt],
                                     sgs[nxt])
                else:
                    def _gnext():
                        pltpu.async_copy(
                            y_hbm.at[gsi.at[1 - qq, k - (_GC - 2)]],
                            bufs[nxt], sgs[nxt])

                    pl.when(g < _NG - 1)(_gnext)

    swait(1, _GC - 2, b2, ss2)
    swait(1, _GC - 1, b3, ss3)

    plsc.subcore_barrier()

    stg = b0.at[pl.ds(0, 80)]

    @pl.when(s < 15)
    def _():
        for k in range(8):
            r0 = s * 640 + k * 80
            pltpu.sync_copy(acc.at[pl.ds(r0, 80)], stg)
            pltpu.sync_copy(stg, out_hbm.at[pl.ds(coff + r0, 80)])

    @pl.when(s == 15)
    def _():
        for k in range(5):
            r0 = 9600 + k * 80
            pltpu.sync_copy(acc.at[pl.ds(r0, 80)], stg)
            pltpu.sync_copy(stg, out_hbm.at[pl.ds(coff + r0, 80)])


def _agg(src4, dst3, y2, dh):
    dt = y2.dtype
    body = functools.partial(_agg_body4, dh, dt)
    return pl.kernel(
        body,
        out_type=jax.ShapeDtypeStruct((2 * _N, dh), dt),
        mesh=_sc_mesh(),
        scratch_types=(
            [pltpu.VMEM_SHARED((_N, dh), dt),
             pltpu.VMEM((2, _GC, _KA), jnp.int32),
             pltpu.VMEM((2, _GC, _KA), jnp.int32)]
            + [pltpu.VMEM((_KA, dh), dt)] * 4
            + [pltpu.SemaphoreType.DMA] * 8
        ),
        compiler_params=pltpu.CompilerParams(use_tc_tiling_on_sc=False),
    )(src4, dst3, y2)


def _tc1(degp, x, w1):
    def body(degp_ref, x_ref, w_ref, dinv_ref, y_ref):
        deg = degp_ref[0] + degp_ref[1] + 1.0
        dinv = lax.rsqrt(deg)
        xw = jnp.dot(x_ref[...], w_ref[...],
                     preferred_element_type=jnp.float32)
        y = (xw * dinv).astype(jnp.bfloat16)
        dinv_ref[...] = dinv
        y_ref[0] = y[:, :128]
        y_ref[1] = y[:, 128:]

    return pl.pallas_call(
        body,
        grid=(_N // _NB,),
        in_specs=[
            pl.BlockSpec((2, _NB, 1), lambda i: (0, i, 0)),
            pl.BlockSpec((_NB, 512), lambda i: (i, 0)),
            pl.BlockSpec((512, 256), lambda i: (0, 0)),
        ],
        out_specs=[
            pl.BlockSpec((_NB, 1), lambda i: (i, 0)),
            pl.BlockSpec((2, _NB, 128), lambda i: (0, i, 0)),
        ],
        out_shape=[
            jax.ShapeDtypeStruct((_N, 1), jnp.float32),
            jax.ShapeDtypeStruct((2, _N, 128), jnp.bfloat16),
        ],
    )(degp, x, w1)


def _tc_mid(agg, y, dinv, b, w, dh_in, dh_out, out_dt=jnp.float32):
    d_in = 2 * dh_in

    def body(agg_ref, y_ref, dinv_ref, b_ref, w_ref, yout_ref):
        dv = dinv_ref[...]
        bv = b_ref[...]
        a0 = agg_ref[0].astype(jnp.float32) + y_ref[0].astype(jnp.float32)
        a1 = agg_ref[1].astype(jnp.float32) + y_ref[1].astype(jnp.float32)
        h0 = jnp.maximum(dv * a0 + bv[:dh_in], 0.0)
        h1 = jnp.maximum(dv * a1 + bv[dh_in:], 0.0)
        h = jnp.concatenate([h0, h1], axis=1)
        xw = jnp.dot(h, w_ref[...], preferred_element_type=jnp.float32)
        yv = (xw * dv).astype(yout_ref.dtype)
        yout_ref[0] = yv[:, :dh_out]
        yout_ref[1] = yv[:, dh_out:]

    return pl.pallas_call(
        body,
        grid=(_N // _NB,),
        in_specs=[
            pl.BlockSpec((2, _NB, dh_in), lambda i: (0, i, 0)),
            pl.BlockSpec((2, _NB, dh_in), lambda i: (0, i, 0)),
            pl.BlockSpec((_NB, 1), lambda i: (i, 0)),
            pl.BlockSpec((d_in,), lambda i: (0,)),
            pl.BlockSpec((d_in, 2 * dh_out), lambda i: (0, 0)),
        ],
        out_specs=pl.BlockSpec((2, _NB, dh_out), lambda i: (0, i, 0)),
        out_shape=jax.ShapeDtypeStruct((2, _N, dh_out), out_dt),
    )(agg, y, dinv, b, w)


def _tc5(agg, y, dinv, b4, batch2, lw1, lb1, lw2, lb2):
    nblk = _N // _NB

    def body(agg_ref, y_ref, dinv_ref, b_ref, batch_ref, lw1_ref, lb1_ref,
             lw2_ref, lb2_ref, out_ref, sums, cnt):
        i = pl.program_id(0)
        dv = dinv_ref[...]
        bv = b_ref[...]
        h0 = dv * (agg_ref[0] + y_ref[0]) + bv[:16]
        h1 = dv * (agg_ref[1] + y_ref[1]) + bv[16:]
        h = jnp.concatenate([h0, h1], axis=1)
        gi = lax.broadcasted_iota(jnp.int32, (_NB, _G), 1)
        oh = (batch_ref[...] == gi).astype(jnp.float32)
        ps = lax.dot_general(oh, h, (((0,), (0,)), ((), ())),
                             preferred_element_type=jnp.float32)
        pc = lax.dot_general(oh, jnp.ones((_NB, 1), jnp.float32),
                             (((0,), (0,)), ((), ())),
                             preferred_element_type=jnp.float32)

        @pl.when(i == 0)
        def _():
            sums[...] = ps
            cnt[...] = pc
            out_ref[...] = jnp.zeros_like(out_ref)

        @pl.when(i > 0)
        def _():
            sums[...] = sums[...] + ps
            cnt[...] = cnt[...] + pc

        @pl.when(i == nblk - 1)
        def _():
            pooled = sums[...] / jnp.maximum(cnt[...], 1.0)
            z = jnp.maximum(
                jnp.dot(pooled, lw1_ref[...],
                        preferred_element_type=jnp.float32) + lb1_ref[...],
                0.0)
            out_ref[...] = jnp.dot(
                z, lw2_ref[...], preferred_element_type=jnp.float32) \
                + lb2_ref[...]

    return pl.pallas_call(
        body,
        grid=(nblk,),
        in_specs=[
            pl.BlockSpec((2, _NB, 16), lambda i: (0, i, 0)),
            pl.BlockSpec((2, _NB, 16), lambda i: (0, i, 0)),
            pl.BlockSpec((_NB, 1), lambda i: (i, 0)),
            pl.BlockSpec((32,), lambda i: (0,)),
            pl.BlockSpec((_NB, 1), lambda i: (i, 0)),
            pl.BlockSpec((32, 16), lambda i: (0, 0)),
            pl.BlockSpec((16,), lambda i: (0,)),
            pl.BlockSpec((16, 2), lambda i: (0, 0)),
            pl.BlockSpec((2,), lambda i: (0,)),
        ],
        out_specs=pl.BlockSpec((_G, 2), lambda i: (0, 0)),
        out_shape=jax.ShapeDtypeStruct((_G, 2), jnp.float32),
        scratch_shapes=[
            pltpu.VMEM((_G, 32), jnp.float32),
            pltpu.VMEM((_G, 1), jnp.float32),
        ],
    )(agg, y, dinv, b4, batch2, lw1, lb1, lw2, lb2)


def kernel(x, edge_index, batch, w1, b1, w2, b2, w3, b3, w4, b4, lw1, lb1,
           lw2, lb2):
    src = edge_index[0]
    dst = edge_index[1]
    src4 = jnp.stack([src, src + _N]).reshape(2, _NS, _NCH, _KA)
    dst3 = dst.reshape(_NS, _NCH, _KA)
    degp = _deg(dst.reshape(_NS, _N // 16, 16)).reshape(2, _N, 1)
    dinv, y1 = _tc1(degp, x, w1)
    agg1 = _agg(src4, dst3, y1.reshape(2 * _N, 128), 128).reshape(2, _N, 128)
    y2 = _tc_mid(agg1, y1, dinv, b1, w2, 128, 64, jnp.bfloat16)
    agg2 = _agg(src4, dst3, y2.reshape(2 * _N, 64), 64).reshape(2, _N, 64)
    y3 = _tc_mid(agg2, y2, dinv, b2, w3, 64, 32, jnp.bfloat16)
    agg3 = _agg(src4, dst3, y3.reshape(2 * _N, 32), 32).reshape(2, _N, 32)
    y4 = _tc_mid(agg3, y3, dinv, b3, w4, 32, 16)
    agg4 = _agg(src4, dst3, y4.reshape(2 * _N, 16), 16).reshape(2, _N, 16)
    return _tc5(agg4, y4, dinv, b4, batch[:, None], lw1, lb1, lw2, lb2)
